# baseline (speedup 1.0000x reference)
import jax
import jax.numpy as jnp
from jax.experimental import pallas as pl


def _final_body(logits_ref, out_ref):
    lg = logits_ref[...]
    m = jnp.max(lg, axis=1, keepdims=True)
    s = jnp.log(jnp.sum(jnp.exp(lg - m), axis=1, keepdims=True))
    out_ref[...] = lg - m - s


def _rgcn_conv(x, src, dst, etype, W_rel, W_root, b):
    n = x.shape[0]
    r = W_rel.shape[0]
    h_all = jnp.einsum('nd,rdh->nrh', x, W_rel)
    m = h_all[src, etype]
    keyc = dst * r + etype
    ones = jnp.ones((src.shape[0],), dtype=x.dtype)
    counts = jax.ops.segment_sum(ones, keyc, num_segments=n * r)
    norm = 1.0 / jnp.maximum(counts, 1.0)
    m = m * norm[keyc][:, None]
    agg = jax.ops.segment_sum(m, dst, num_segments=n)
    return agg + x @ W_root + b


def _batchnorm(h, gamma, beta):
    mu = jnp.mean(h, axis=0)
    var = jnp.var(h, axis=0)
    return (h - mu) / jnp.sqrt(var + 1e-5) * gamma + beta


def kernel(x, edge_index, edge_attr, W_rel0, W_root0, b0, gamma0, beta0,
           W_rel1, W_root1, b1, gamma1, beta1, Wf, bf):
    src = edge_index[0]
    dst = edge_index[1]
    h = _rgcn_conv(x, src, dst, edge_attr, W_rel0, W_root0, b0)
    h = _batchnorm(h, gamma0, beta0)
    h = jax.nn.relu(h)
    h = _rgcn_conv(h, src, dst, edge_attr, W_rel1, W_root1, b1)
    h = _batchnorm(h, gamma1, beta1)
    h = jax.nn.relu(h)
    logits = h @ Wf + bf
    n = logits.shape[0]
    blk = 2000
    return pl.pallas_call(
        _final_body,
        out_shape=jax.ShapeDtypeStruct(logits.shape, logits.dtype),
        grid=(n // blk,),
        in_specs=[pl.BlockSpec((blk, logits.shape[1]), lambda i: (i, 0))],
        out_specs=pl.BlockSpec((blk, logits.shape[1]), lambda i: (i, 0)),
    )(logits)


# speedup vs baseline: 6.9473x; 6.9473x over previous
import functools

import jax
import jax.numpy as jnp
from jax import lax
from jax.experimental import pallas as pl
from jax.experimental.pallas import tpu as pltpu
from jax.experimental.pallas import tpu_sc as plsc

N = 100000
E = 1600000
D = 128
H = 16
R = 16
C = 40

NC = 2
NS = 16
LANES = 16

SUPER = 1024
CHUNK = 128
E_PAD = 1605632
PADN = E_PAD - E
CN = 1632000
ACC_ROWS = 100352

_BLK = 1000
_GRID = N // _BLK


def _mm2_body(x_ref, w1_ref, w2_ref, o1_ref, o2_ref):
    xb = x_ref[...]
    o1_ref[...] = jnp.dot(xb, w1_ref[...], preferred_element_type=jnp.float32)
    o2_ref[...] = jnp.dot(xb, w2_ref[...], preferred_element_type=jnp.float32)


def _tc_mm2(x, w1, w2):
    k = x.shape[1]
    return pl.pallas_call(
        _mm2_body,
        grid=(_GRID,),
        in_specs=[
            pl.BlockSpec((_BLK, k), lambda i: (i, 0)),
            pl.BlockSpec(w1.shape, lambda i: (0, 0)),
            pl.BlockSpec(w2.shape, lambda i: (0, 0)),
        ],
        out_specs=[
            pl.BlockSpec((_BLK, w1.shape[1]), lambda i: (i, 0)),
            pl.BlockSpec((_BLK, w2.shape[1]), lambda i: (i, 0)),
        ],
        out_shape=[
            jax.ShapeDtypeStruct((N, w1.shape[1]), jnp.float32),
            jax.ShapeDtypeStruct((N, w2.shape[1]), jnp.float32),
        ],
    )(x, w1, w2)


def _stats_body(a0_ref, a1_ref, xr_ref, b_ref, h_ref, s_ref):
    h = a0_ref[...] + a1_ref[...] + xr_ref[...] + b_ref[...]
    h_ref[...] = h
    upd = jnp.concatenate([jnp.sum(h, 0)[None], jnp.sum(h * h, 0)[None]], 0)

    @pl.when(pl.program_id(0) == 0)
    def _():
        s_ref[...] = jnp.zeros_like(s_ref)

    s_ref[...] += upd


def _tc_stats(a0, a1, xr, b):
    return pl.pallas_call(
        _stats_body,
        grid=(_GRID,),
        in_specs=[
            pl.BlockSpec((_BLK, H), lambda i: (i, 0)),
            pl.BlockSpec((_BLK, H), lambda i: (i, 0)),
            pl.BlockSpec((_BLK, H), lambda i: (i, 0)),
            pl.BlockSpec((1, H), lambda i: (0, 0)),
        ],
        out_specs=[
            pl.BlockSpec((_BLK, H), lambda i: (i, 0)),
            pl.BlockSpec((2, H), lambda i: (0, 0)),
        ],
        out_shape=[
            jax.ShapeDtypeStruct((N, H), jnp.float32),
            jax.ShapeDtypeStruct((2, H), jnp.float32),
        ],
    )(a0, a1, xr, b)


def _bn_relu(h, st, g, be):
    mu = st[0:1, :] * (1.0 / N)
    var = st[1:2, :] * (1.0 / N) - mu * mu
    inv = lax.rsqrt(var + 1e-5)
    return jnp.maximum((h - mu) * inv * g + be, 0.0)


def _norm_mm_body(h_ref, st_ref, g_ref, be_ref, w1_ref, w2_ref, o1_ref, o2_ref):
    hn = _bn_relu(h_ref[...], st_ref[...], g_ref[...], be_ref[...])
    o1_ref[...] = jnp.dot(hn, w1_ref[...], preferred_element_type=jnp.float32)
    o2_ref[...] = jnp.dot(hn, w2_ref[...], preferred_element_type=jnp.float32)


def _tc_norm_mm(h, st, g, be, w1, w2):
    return pl.pallas_call(
        _norm_mm_body,
        grid=(_GRID,),
        in_specs=[
            pl.BlockSpec((_BLK, H), lambda i: (i, 0)),
            pl.BlockSpec((2, H), lambda i: (0, 0)),
            pl.BlockSpec((1, H), lambda i: (0, 0)),
            pl.BlockSpec((1, H), lambda i: (0, 0)),
            pl.BlockSpec(w1.shape, lambda i: (0, 0)),
            pl.BlockSpec(w2.shape, lambda i: (0, 0)),
        ],
        out_specs=[
            pl.BlockSpec((_BLK, w1.shape[1]), lambda i: (i, 0)),
            pl.BlockSpec((_BLK, w2.shape[1]), lambda i: (i, 0)),
        ],
        out_shape=[
            jax.ShapeDtypeStruct((N, w1.shape[1]), jnp.float32),
            jax.ShapeDtypeStruct((N, w2.shape[1]), jnp.float32),
        ],
    )(h, st, g, be, w1, w2)


def _final_body(h_ref, st_ref, g_ref, be_ref, wf_ref, bf_ref, o_ref):
    hn = _bn_relu(h_ref[...], st_ref[...], g_ref[...], be_ref[...])
    lg = jnp.dot(hn, wf_ref[...], preferred_element_type=jnp.float32) + bf_ref[...]
    m = jnp.max(lg, axis=1, keepdims=True)
    s = jnp.log(jnp.sum(jnp.exp(lg - m), axis=1, keepdims=True))
    o_ref[...] = lg - m - s


def _tc_final(h, st, g, be, wf, bf):
    return pl.pallas_call(
        _final_body,
        grid=(_GRID,),
        in_specs=[
            pl.BlockSpec((_BLK, H), lambda i: (i, 0)),
            pl.BlockSpec((2, H), lambda i: (0, 0)),
            pl.BlockSpec((1, H), lambda i: (0, 0)),
            pl.BlockSpec((1, H), lambda i: (0, 0)),
            pl.BlockSpec((H, C), lambda i: (0, 0)),
            pl.BlockSpec((1, C), lambda i: (0, 0)),
        ],
        out_specs=pl.BlockSpec((_BLK, C), lambda i: (i, 0)),
        out_shape=jax.ShapeDtypeStruct((N, C), jnp.float32),
    )(h, st, g, be, wf, bf)


_MESH = plsc.VectorSubcoreMesh(core_axis_name="c", subcore_axis_name="s")
_SC_PARAMS = pltpu.CompilerParams(use_tc_tiling_on_sc=False)


def _keys_into(d_ref, e_ref, k2_ref):
    def body(i, _):
        dv = d_ref[pl.ds(16 * i, 16)]
        ev = e_ref[pl.ds(16 * i, 16)]
        cc = i // 8
        off = (i % 8) * 16
        k2_ref[cc, pl.ds(off, 16)] = dv * R + ev
        return 0
    lax.fori_loop(0, SUPER // 16, body, 0, unroll=4)


def _wnorm_body(dst_hbm, et_hbm, wn_hbm, counts_sh, d_buf, e_buf, k2, ones,
                cbuf, wbuf):
    s = lax.axis_index("s")
    c = lax.axis_index("c")

    def zb(i, _):
        cbuf[pl.ds(16 * i, 16)] = jnp.zeros((16,), jnp.float32)
        return 0
    lax.fori_loop(0, 125, zb, 0, unroll=4)

    def z2(j, _):
        pltpu.sync_copy(cbuf, counts_sh.at[pl.ds(s * 102000 + 2000 * j, 2000)])
        return 0
    lax.fori_loop(0, 51, z2, 0)

    def ob(i, _):
        ones[pl.ds(16 * i, 16)] = jnp.ones((16,), jnp.float32)
        return 0
    lax.fori_loop(0, CHUNK // 16, ob, 0)
    plsc.subcore_barrier()

    per_tile = E_PAD // NS

    def count_super(j, _):
        base = s * per_tile + j * SUPER
        pltpu.sync_copy(dst_hbm.at[pl.ds(base, SUPER)], d_buf)
        pltpu.sync_copy(et_hbm.at[pl.ds(base, SUPER)], e_buf)
        _keys_into(d_buf, e_buf, k2)

        def sc1(cc, _):
            pltpu.sync_copy(ones, counts_sh.at[k2.at[cc]], add=True)
            return 0
        lax.fori_loop(0, SUPER // CHUNK, sc1, 0)
        return 0
    lax.fori_loop(0, per_tile // SUPER, count_super, 0)
    plsc.subcore_barrier()

    def inv_chunk(j, _):
        off = s * 102000 + 2000 * j
        pltpu.sync_copy(counts_sh.at[pl.ds(off, 2000)], cbuf)

        def iv(i, _):
            v = cbuf[pl.ds(16 * i, 16)]
            cbuf[pl.ds(16 * i, 16)] = 1.0 / jnp.maximum(v, 1.0)
            return 0
        lax.fori_loop(0, 125, iv, 0, unroll=4)
        pltpu.sync_copy(cbuf, counts_sh.at[pl.ds(off, 2000)])
        return 0
    lax.fori_loop(0, 51, inv_chunk, 0)
    plsc.subcore_barrier()

    half = E_PAD // NC
    per_tile_h = half // NS

    def wn_super(j, _):
        base = c * half + s * per_tile_h + j * SUPER
        pltpu.sync_copy(dst_hbm.at[pl.ds(base, SUPER)], d_buf)
        pltpu.sync_copy(et_hbm.at[pl.ds(base, SUPER)], e_buf)
        _keys_into(d_buf, e_buf, k2)

        def g1(cc, _):
            pltpu.sync_copy(counts_sh.at[k2.at[cc]],
                            wbuf.at[pl.ds(CHUNK * cc, CHUNK)])
            return 0
        lax.fori_loop(0, SUPER // CHUNK, g1, 0)
        pltpu.sync_copy(wbuf, wn_hbm.at[pl.ds(base, SUPER)])
        return 0
    lax.fori_loop(0, per_tile_h // SUPER, wn_super, 0)


def _sc_wnorm(dstp, etp):
    fn = pl.kernel(
        _wnorm_body,
        out_type=jax.ShapeDtypeStruct((E_PAD,), jnp.float32),
        mesh=_MESH,
        scratch_types=[
            pltpu.VMEM_SHARED((CN,), jnp.float32),
            pltpu.VMEM((SUPER,), jnp.int32),
            pltpu.VMEM((SUPER,), jnp.int32),
            pltpu.VMEM((SUPER // CHUNK, CHUNK), jnp.int32),
            pltpu.VMEM((CHUNK,), jnp.float32),
            pltpu.VMEM((2000,), jnp.float32),
            pltpu.VMEM((SUPER,), jnp.float32),
        ],
        compiler_params=_SC_PARAMS,
    )
    return fn(dstp, etp)


def _msg_body(tab_hbm, src_hbm, et_hbm, wn_hbm, dst_hbm, agg_hbm,
              acc_sh, s_buf, e_buf, d_buf, w_buf, g_buf, d2, rows, zrow, sem):
    s = lax.axis_index("s")
    c = lax.axis_index("c")

    def zr(i, _):
        zrow[i, :] = jnp.zeros((16,), jnp.float32)
        return 0
    lax.fori_loop(0, CHUNK, zr, 0, unroll=4)

    def z2(j, _):
        pltpu.sync_copy(zrow, acc_sh.at[pl.ds(s * (ACC_ROWS // NS) + CHUNK * j, CHUNK), :])
        return 0
    lax.fori_loop(0, ACC_ROWS // NS // CHUNK, z2, 0)
    plsc.subcore_barrier()

    half = E_PAD // NC
    per_tile = half // NS

    def msg_super(j, _):
        base = c * half + s * per_tile + j * SUPER
        pltpu.sync_copy(src_hbm.at[pl.ds(base, SUPER)], s_buf)
        pltpu.sync_copy(et_hbm.at[pl.ds(base, SUPER)], e_buf)
        pltpu.sync_copy(dst_hbm.at[pl.ds(base, SUPER)], d_buf)
        pltpu.sync_copy(wn_hbm.at[pl.ds(base, SUPER)], w_buf)

        def kb(i, _):
            sv = s_buf[pl.ds(16 * i, 16)]
            ev = e_buf[pl.ds(16 * i, 16)]
            g_buf[pl.ds(16 * i, 16)] = sv * R + ev
            cc = i // 8
            off = (i % 8) * 16
            d2[cc, pl.ds(off, 16)] = d_buf[pl.ds(16 * i, 16)]
            return 0
        lax.fori_loop(0, SUPER // 16, kb, 0, unroll=4)

        descs = []
        for cc in range(SUPER // CHUNK):
            descs.append(pltpu.async_copy(
                tab_hbm.at[g_buf.at[pl.ds(CHUNK * cc, CHUNK)]],
                rows.at[pl.ds(CHUNK * cc, CHUNK), :], sem))
        for dsc in descs:
            dsc.wait()

        def sc_scale(i, _):
            wv = w_buf[pl.ds(16 * i, 16)]
            for k in range(16):
                rows[16 * i + k, :] = rows[16 * i + k, :] * wv[k]
            return 0
        lax.fori_loop(0, SUPER // 16, sc_scale, 0)

        for cc in range(SUPER // CHUNK):
            pltpu.sync_copy(rows.at[pl.ds(CHUNK * cc, CHUNK), :],
                            acc_sh.at[d2.at[cc]], add=True)
        return 0
    lax.fori_loop(0, per_tile // SUPER, msg_super, 0)
    plsc.subcore_barrier()

    def wo(j, _):
        k = j * NS + s

        @pl.when(k < N // 1000)
        def _():
            r0 = k * 1000
            pltpu.sync_copy(acc_sh.at[pl.ds(r0, 1000), :],
                            agg_hbm.at[c, pl.ds(r0, 1000), :])
        return 0
    lax.fori_loop(0, (N // 1000 + NS - 1) // NS, wo, 0)


def _sc_msg(table, srcp, etp, wnorm, dstp):
    fn = pl.kernel(
        _msg_body,
        out_type=jax.ShapeDtypeStruct((NC, N, H), jnp.float32),
        mesh=_MESH,
        scratch_types=[
            pltpu.VMEM_SHARED((ACC_ROWS, H), jnp.float32),
            pltpu.VMEM((SUPER,), jnp.int32),
            pltpu.VMEM((SUPER,), jnp.int32),
            pltpu.VMEM((SUPER,), jnp.int32),
            pltpu.VMEM((SUPER,), jnp.float32),
            pltpu.VMEM((SUPER,), jnp.int32),
            pltpu.VMEM((SUPER // CHUNK, CHUNK), jnp.int32),
            pltpu.VMEM((SUPER, H), jnp.float32),
            pltpu.VMEM((CHUNK, H), jnp.float32),
            pltpu.SemaphoreType.DMA,
        ],
        compiler_params=_SC_PARAMS,
    )
    return fn(table, srcp, etp, wnorm, dstp)


def kernel(x, edge_index, edge_attr, W_rel0, W_root0, b0, gamma0, beta0,
           W_rel1, W_root1, b1, gamma1, beta1, Wf, bf):
    src = edge_index[0]
    dst = edge_index[1]
    et = edge_attr
    zpad = jnp.zeros((PADN,), jnp.int32)
    srcp = jnp.concatenate([src, zpad])
    etp = jnp.concatenate([et, zpad])
    dstp = jnp.concatenate([dst, jnp.full((PADN,), N, jnp.int32)])

    w1f0 = jnp.transpose(W_rel0, (1, 0, 2)).reshape(D, R * H)
    w1f1 = jnp.transpose(W_rel1, (1, 0, 2)).reshape(H, R * H)

    h_all0, xr0 = _tc_mm2(x, w1f0, W_root0)
    wnorm = _sc_wnorm(dstp, etp)
    agg0 = _sc_msg(h_all0.reshape(N * R, H), srcp, etp, wnorm, dstp)
    h0, st0 = _tc_stats(agg0[0], agg0[1], xr0, b0.reshape(1, H))
    h_all1, hr1 = _tc_norm_mm(h0, st0, gamma0.reshape(1, H), beta0.reshape(1, H),
                              w1f1, W_root1)
    agg1 = _sc_msg(h_all1.reshape(N * R, H), srcp, etp, wnorm, dstp)
    h1, st1 = _tc_stats(agg1[0], agg1[1], hr1, b1.reshape(1, H))
    return _tc_final(h1, st1, gamma1.reshape(1, H), beta1.reshape(1, H),
                     Wf, bf.reshape(1, C))


# speedup vs baseline: 7.8040x; 1.1233x over previous
import functools

import jax
import jax.numpy as jnp
from jax import lax
from jax.experimental import pallas as pl
from jax.experimental.pallas import tpu as pltpu
from jax.experimental.pallas import tpu_sc as plsc

N = 100000
E = 1600000
D = 128
H = 16
R = 16
C = 40

NC = 2
NS = 16
LANES = 16

SUPER = 512
CHUNK = 128
E_PAD = 1605632
PADN = E_PAD - E
CN = 1632000
ACC_ROWS = 100352

_BLK = 1000
_GRID = N // _BLK


def _mm2_body(x_ref, w1_ref, w2_ref, o1_ref, o2_ref):
    xb = x_ref[...]
    o1_ref[0] = jnp.dot(xb, w1_ref[...], preferred_element_type=jnp.float32)
    o2_ref[...] = jnp.dot(xb, w2_ref[...], preferred_element_type=jnp.float32)


def _tc_mm2(x, w1, w2):
    k = x.shape[1]
    return pl.pallas_call(
        _mm2_body,
        grid=(_GRID, 2),
        in_specs=[
            pl.BlockSpec((_BLK, k), lambda i, j: (i, 0)),
            pl.BlockSpec((k, 128), lambda i, j: (0, j)),
            pl.BlockSpec((k, H), lambda i, j: (0, 0)),
        ],
        out_specs=[
            pl.BlockSpec((1, _BLK, 128), lambda i, j: (j, i, 0)),
            pl.BlockSpec((_BLK, H), lambda i, j: (i, 0)),
        ],
        out_shape=[
            jax.ShapeDtypeStruct((2, N, 128), jnp.float32),
            jax.ShapeDtypeStruct((N, H), jnp.float32),
        ],
    )(x, w1, w2)


def _stats_body(a0_ref, a1_ref, xr_ref, b_ref, h_ref, s_ref):
    h = a0_ref[...] + a1_ref[...] + xr_ref[...] + b_ref[...]
    h_ref[...] = h
    upd = jnp.concatenate([jnp.sum(h, 0)[None], jnp.sum(h * h, 0)[None]], 0)

    @pl.when(pl.program_id(0) == 0)
    def _():
        s_ref[...] = jnp.zeros_like(s_ref)

    s_ref[...] += upd


def _tc_stats(a0, a1, xr, b):
    return pl.pallas_call(
        _stats_body,
        grid=(_GRID,),
        in_specs=[
            pl.BlockSpec((_BLK, H), lambda i: (i, 0)),
            pl.BlockSpec((_BLK, H), lambda i: (i, 0)),
            pl.BlockSpec((_BLK, H), lambda i: (i, 0)),
            pl.BlockSpec((1, H), lambda i: (0, 0)),
        ],
        out_specs=[
            pl.BlockSpec((_BLK, H), lambda i: (i, 0)),
            pl.BlockSpec((2, H), lambda i: (0, 0)),
        ],
        out_shape=[
            jax.ShapeDtypeStruct((N, H), jnp.float32),
            jax.ShapeDtypeStruct((2, H), jnp.float32),
        ],
    )(a0, a1, xr, b)


def _bn_relu(h, st, g, be):
    mu = st[0:1, :] * (1.0 / N)
    var = st[1:2, :] * (1.0 / N) - mu * mu
    inv = lax.rsqrt(var + 1e-5)
    return jnp.maximum((h - mu) * inv * g + be, 0.0)


def _norm_mm_body(h_ref, st_ref, g_ref, be_ref, w1_ref, w2_ref, o1_ref, o2_ref):
    hn = _bn_relu(h_ref[...], st_ref[...], g_ref[...], be_ref[...])
    o1_ref[0] = jnp.dot(hn, w1_ref[...], preferred_element_type=jnp.float32)
    o2_ref[...] = jnp.dot(hn, w2_ref[...], preferred_element_type=jnp.float32)


def _tc_norm_mm(h, st, g, be, w1, w2):
    return pl.pallas_call(
        _norm_mm_body,
        grid=(_GRID, 2),
        in_specs=[
            pl.BlockSpec((_BLK, H), lambda i, j: (i, 0)),
            pl.BlockSpec((2, H), lambda i, j: (0, 0)),
            pl.BlockSpec((1, H), lambda i, j: (0, 0)),
            pl.BlockSpec((1, H), lambda i, j: (0, 0)),
            pl.BlockSpec((H, 128), lambda i, j: (0, j)),
            pl.BlockSpec((H, H), lambda i, j: (0, 0)),
        ],
        out_specs=[
            pl.BlockSpec((1, _BLK, 128), lambda i, j: (j, i, 0)),
            pl.BlockSpec((_BLK, H), lambda i, j: (i, 0)),
        ],
        out_shape=[
            jax.ShapeDtypeStruct((2, N, 128), jnp.float32),
            jax.ShapeDtypeStruct((N, H), jnp.float32),
        ],
    )(h, st, g, be, w1, w2)


def _final_body(h_ref, st_ref, g_ref, be_ref, wf_ref, bf_ref, o_ref):
    hn = _bn_relu(h_ref[...], st_ref[...], g_ref[...], be_ref[...])
    lg = jnp.dot(hn, wf_ref[...], preferred_element_type=jnp.float32) + bf_ref[...]
    m = jnp.max(lg, axis=1, keepdims=True)
    s = jnp.log(jnp.sum(jnp.exp(lg - m), axis=1, keepdims=True))
    o_ref[...] = lg - m - s


def _tc_final(h, st, g, be, wf, bf):
    return pl.pallas_call(
        _final_body,
        grid=(_GRID,),
        in_specs=[
            pl.BlockSpec((_BLK, H), lambda i: (i, 0)),
            pl.BlockSpec((2, H), lambda i: (0, 0)),
            pl.BlockSpec((1, H), lambda i: (0, 0)),
            pl.BlockSpec((1, H), lambda i: (0, 0)),
            pl.BlockSpec((H, C), lambda i: (0, 0)),
            pl.BlockSpec((1, C), lambda i: (0, 0)),
        ],
        out_specs=pl.BlockSpec((_BLK, C), lambda i: (i, 0)),
        out_shape=jax.ShapeDtypeStruct((N, C), jnp.float32),
    )(h, st, g, be, wf, bf)


_MESH = plsc.VectorSubcoreMesh(core_axis_name="c", subcore_axis_name="s")
_SC_PARAMS = pltpu.CompilerParams(use_tc_tiling_on_sc=False)


def _keys_into(d_ref, e_ref, k2_ref):
    def body(i, _):
        dv = d_ref[pl.ds(16 * i, 16)]
        ev = e_ref[pl.ds(16 * i, 16)]
        cc = i // 8
        off = (i % 8) * 16
        k2_ref[cc, pl.ds(off, 16)] = dv * R + ev
        return 0
    lax.fori_loop(0, SUPER // 16, body, 0, unroll=4)


def _wnorm_body(dst_hbm, et_hbm, wn_hbm, counts_sh, d_buf, e_buf, k2, ones,
                cbuf, wbuf):
    s = lax.axis_index("s")
    c = lax.axis_index("c")

    def zb(i, _):
        cbuf[pl.ds(16 * i, 16)] = jnp.zeros((16,), jnp.float32)
        return 0
    lax.fori_loop(0, 125, zb, 0, unroll=4)

    def z2(j, _):
        pltpu.sync_copy(cbuf, counts_sh.at[pl.ds(s * 102000 + 2000 * j, 2000)])
        return 0
    lax.fori_loop(0, 51, z2, 0)

    def ob(i, _):
        ones[pl.ds(16 * i, 16)] = jnp.ones((16,), jnp.float32)
        return 0
    lax.fori_loop(0, CHUNK // 16, ob, 0)
    plsc.subcore_barrier()

    per_tile = E_PAD // NS

    def count_super(j, _):
        base = s * per_tile + j * SUPER
        pltpu.sync_copy(dst_hbm.at[pl.ds(base, SUPER)], d_buf)
        pltpu.sync_copy(et_hbm.at[pl.ds(base, SUPER)], e_buf)
        _keys_into(d_buf, e_buf, k2)

        def sc1(cc, _):
            pltpu.sync_copy(ones, counts_sh.at[k2.at[cc]], add=True)
            return 0
        lax.fori_loop(0, SUPER // CHUNK, sc1, 0)
        return 0
    lax.fori_loop(0, per_tile // SUPER, count_super, 0)
    plsc.subcore_barrier()

    def inv_chunk(j, _):
        off = s * 102000 + 2000 * j
        pltpu.sync_copy(counts_sh.at[pl.ds(off, 2000)], cbuf)

        def iv(i, _):
            v = cbuf[pl.ds(16 * i, 16)]
            cbuf[pl.ds(16 * i, 16)] = 1.0 / jnp.maximum(v, 1.0)
            return 0
        lax.fori_loop(0, 125, iv, 0, unroll=4)
        pltpu.sync_copy(cbuf, counts_sh.at[pl.ds(off, 2000)])
        return 0
    lax.fori_loop(0, 51, inv_chunk, 0)
    plsc.subcore_barrier()

    half = E_PAD // NC
    per_tile_h = half // NS

    def wn_super(j, _):
        base = c * half + s * per_tile_h + j * SUPER
        pltpu.sync_copy(dst_hbm.at[pl.ds(base, SUPER)], d_buf)
        pltpu.sync_copy(et_hbm.at[pl.ds(base, SUPER)], e_buf)
        _keys_into(d_buf, e_buf, k2)

        def g1(cc, _):
            pltpu.sync_copy(counts_sh.at[k2.at[cc]],
                            wbuf.at[pl.ds(CHUNK * cc, CHUNK)])
            return 0
        lax.fori_loop(0, SUPER // CHUNK, g1, 0)
        pltpu.sync_copy(wbuf, wn_hbm.at[pl.ds(base, SUPER)])
        return 0
    lax.fori_loop(0, per_tile_h // SUPER, wn_super, 0)


def _sc_wnorm(dstp, etp):
    fn = pl.kernel(
        _wnorm_body,
        out_type=jax.ShapeDtypeStruct((E_PAD,), jnp.float32),
        mesh=_MESH,
        scratch_types=[
            pltpu.VMEM_SHARED((CN,), jnp.float32),
            pltpu.VMEM((SUPER,), jnp.int32),
            pltpu.VMEM((SUPER,), jnp.int32),
            pltpu.VMEM((SUPER // CHUNK, CHUNK), jnp.int32),
            pltpu.VMEM((CHUNK,), jnp.float32),
            pltpu.VMEM((2000,), jnp.float32),
            pltpu.VMEM((SUPER,), jnp.float32),
        ],
        compiler_params=_SC_PARAMS,
    )
    return fn(dstp, etp)


def _msg_body(tab_hbm, src_hbm, et_hbm, wn_hbm, dst_hbm, agg_hbm,
              acc_sh,
              s_buf0, e_buf0, d_buf0, w_buf0, g_buf0, d20, rows0,
              s_buf1, e_buf1, d_buf1, w_buf1, g_buf1, d21, rows1,
              zrow, sem_st0, sem_st1, sem_g0, sem_g1, sem_sc0, sem_sc1):
    s = lax.axis_index("s")
    c = lax.axis_index("c")
    sb = (s_buf0, s_buf1)
    eb = (e_buf0, e_buf1)
    db = (d_buf0, d_buf1)
    wb = (w_buf0, w_buf1)
    gb = (g_buf0, g_buf1)
    d2 = (d20, d21)
    rows = (rows0, rows1)
    sem_st = (sem_st0, sem_st1)
    sem_g = (sem_g0, sem_g1)
    sem_sc = (sem_sc0, sem_sc1)
    nchunk = SUPER // CHUNK

    def zr(i, _):
        zrow[i, :] = jnp.zeros((16,), jnp.float32)
        return 0
    lax.fori_loop(0, CHUNK, zr, 0, unroll=4)

    def z2(j, _):
        pltpu.sync_copy(zrow, acc_sh.at[pl.ds(s * (ACC_ROWS // NS) + CHUNK * j, CHUNK), :])
        return 0
    lax.fori_loop(0, ACC_ROWS // NS // CHUNK, z2, 0)
    plsc.subcore_barrier()

    half = E_PAD // NC
    per_tile = half // NS
    nsup = per_tile // SUPER
    tile_base = c * half + s * per_tile

    def start_streams(p, base):
        pltpu.async_copy(src_hbm.at[pl.ds(base, SUPER)], sb[p], sem_st[p])
        pltpu.async_copy(et_hbm.at[pl.ds(base, SUPER)], eb[p], sem_st[p])
        pltpu.async_copy(dst_hbm.at[pl.ds(base, SUPER)], db[p], sem_st[p])
        pltpu.async_copy(wn_hbm.at[pl.ds(base, SUPER)], wb[p], sem_st[p])

    def wait_streams(p):
        pltpu.make_async_copy(src_hbm.at[pl.ds(0, SUPER)], sb[p], sem_st[p]).wait()
        pltpu.make_async_copy(et_hbm.at[pl.ds(0, SUPER)], eb[p], sem_st[p]).wait()
        pltpu.make_async_copy(dst_hbm.at[pl.ds(0, SUPER)], db[p], sem_st[p]).wait()
        pltpu.make_async_copy(wn_hbm.at[pl.ds(0, SUPER)], wb[p], sem_st[p]).wait()

    def wait_scatters(p):
        for cc in range(nchunk):
            pltpu.make_async_copy(rows[p].at[pl.ds(CHUNK * cc, CHUNK), :],
                                  acc_sh.at[d2[p].at[cc]], sem_sc[p]).wait()

    def process(p):
        def kb(i, _):
            sv = sb[p][pl.ds(16 * i, 16)]
            ev = eb[p][pl.ds(16 * i, 16)]
            gb[p][pl.ds(16 * i, 16)] = (sv * 8 + (ev & 7)
                                        + (ev >> 3) * (8 * N))
            cc = i // 8
            off = (i % 8) * 16
            d2[p][cc, pl.ds(off, 16)] = db[p][pl.ds(16 * i, 16)]
            return 0
        lax.fori_loop(0, SUPER // 16, kb, 0, unroll=4)

        descs = []
        for cc in range(nchunk):
            descs.append(pltpu.async_copy(
                tab_hbm.at[gb[p].at[pl.ds(CHUNK * cc, CHUNK)]],
                rows[p].at[pl.ds(CHUNK * cc, CHUNK), :], sem_g[p]))
        for dsc in descs:
            dsc.wait()

        def sc_scale(i, _):
            wv = wb[p][pl.ds(16 * i, 16)]
            for k in range(16):
                rows[p][16 * i + k, :] = rows[p][16 * i + k, :] * wv[k]
            return 0
        lax.fori_loop(0, SUPER // 16, sc_scale, 0)

        for cc in range(nchunk):
            pltpu.async_copy(rows[p].at[pl.ds(CHUNK * cc, CHUNK), :],
                             acc_sh.at[d2[p].at[cc]], sem_sc[p], add=True)

    start_streams(0, tile_base)

    def pair(jj, _):
        j0 = 2 * jj
        wait_streams(0)
        start_streams(1, tile_base + (j0 + 1) * SUPER)

        @pl.when(jj > 0)
        def _():
            wait_scatters(0)
        process(0)

        wait_streams(1)

        @pl.when(j0 + 2 < nsup)
        def _():
            start_streams(0, tile_base + (j0 + 2) * SUPER)

        @pl.when(jj > 0)
        def _():
            wait_scatters(1)
        process(1)
        return 0
    lax.fori_loop(0, nsup // 2, pair, 0)
    wait_scatters(0)
    wait_scatters(1)
    plsc.subcore_barrier()

    def wo(j, _):
        k = j * NS + s

        @pl.when(k < N // 1000)
        def _():
            r0 = k * 1000
            pltpu.sync_copy(acc_sh.at[pl.ds(r0, 1000), :],
                            agg_hbm.at[c, pl.ds(r0, 1000), :])
        return 0
    lax.fori_loop(0, (N // 1000 + NS - 1) // NS, wo, 0)


def _sc_msg(table, srcp, etp, wnorm, dstp):
    per_parity = [
        pltpu.VMEM((SUPER,), jnp.int32),
        pltpu.VMEM((SUPER,), jnp.int32),
        pltpu.VMEM((SUPER,), jnp.int32),
        pltpu.VMEM((SUPER,), jnp.float32),
        pltpu.VMEM((SUPER,), jnp.int32),
        pltpu.VMEM((SUPER // CHUNK, CHUNK), jnp.int32),
        pltpu.VMEM((SUPER, H), jnp.float32),
    ]
    fn = pl.kernel(
        _msg_body,
        out_type=jax.ShapeDtypeStruct((NC, N, H), jnp.float32),
        mesh=_MESH,
        scratch_types=(
            [pltpu.VMEM_SHARED((ACC_ROWS, H), jnp.float32)]
            + per_parity + per_parity
            + [pltpu.VMEM((CHUNK, H), jnp.float32)]
            + [pltpu.SemaphoreType.DMA] * 6
        ),
        compiler_params=_SC_PARAMS,
    )
    return fn(table, srcp, etp, wnorm, dstp)


def kernel(x, edge_index, edge_attr, W_rel0, W_root0, b0, gamma0, beta0,
           W_rel1, W_root1, b1, gamma1, beta1, Wf, bf):
    src = edge_index[0]
    dst = edge_index[1]
    et = edge_attr
    zpad = jnp.zeros((PADN,), jnp.int32)
    srcp = jnp.concatenate([src, zpad])
    etp = jnp.concatenate([et, zpad])
    dstp = jnp.concatenate([dst, jnp.full((PADN,), N, jnp.int32)])

    w1f0 = jnp.transpose(W_rel0, (1, 0, 2)).reshape(D, R * H)
    w1f1 = jnp.transpose(W_rel1, (1, 0, 2)).reshape(H, R * H)

    tab0, xr0 = _tc_mm2(x, w1f0, W_root0)
    wnorm = _sc_wnorm(dstp, etp)
    agg0 = _sc_msg(tab0.reshape(N * R, H), srcp, etp, wnorm, dstp)
    h0, st0 = _tc_stats(agg0[0], agg0[1], xr0, b0.reshape(1, H))
    tab1, hr1 = _tc_norm_mm(h0, st0, gamma0.reshape(1, H), beta0.reshape(1, H),
                            w1f1, W_root1)
    agg1 = _sc_msg(tab1.reshape(N * R, H), srcp, etp, wnorm, dstp)
    h1, st1 = _tc_stats(agg1[0], agg1[1], hr1, b1.reshape(1, H))
    return _tc_final(h1, st1, gamma1.reshape(1, H), beta1.reshape(1, H),
                     Wf, bf.reshape(1, C))


# speedup vs baseline: 9.6741x; 1.2396x over previous
import functools

import jax
import jax.numpy as jnp
from jax import lax
from jax.experimental import pallas as pl
from jax.experimental.pallas import tpu as pltpu
from jax.experimental.pallas import tpu_sc as plsc

N = 100000
E = 1600000
D = 128
H = 16
R = 16
C = 40

NC = 2
NS = 16
LANES = 16

SUPER = 512
CHUNK = 128
E_PAD = 1605632
PADN = E_PAD - E
CN = 1632000
ACC_ROWS = 100352

NP = 100352
_BLK = 1024
_BLK8 = _BLK // 8
MN8 = N // 8
MN8P = NP // 8
_GRID = NP // _BLK


def _unpack_mask(big):
    q = lax.broadcasted_iota(jnp.int32, (_BLK, 128), 1) // 16
    rm = lax.broadcasted_iota(jnp.int32, (_BLK, 128), 0) % 8
    return big * (q == rm).astype(jnp.float32)


def _unpack(hp):
    r8 = lax.broadcasted_iota(jnp.int32, (_BLK, _BLK8), 0) // 8
    mcol = lax.broadcasted_iota(jnp.int32, (_BLK, _BLK8), 1)
    t = (r8 == mcol).astype(jnp.float32)
    big = jnp.dot(t, hp, preferred_element_type=jnp.float32)
    return _unpack_mask(big)


def _pack(big):
    masked = _unpack_mask(big)
    r8 = lax.broadcasted_iota(jnp.int32, (_BLK8, _BLK), 1) // 8
    mrow = lax.broadcasted_iota(jnp.int32, (_BLK8, _BLK), 0)
    p = (r8 == mrow).astype(jnp.float32)
    return jnp.dot(p, masked, preferred_element_type=jnp.float32)


def _chanfold(st):
    a = lax.broadcasted_iota(jnp.int32, (128, 128), 0) % 16
    b = lax.broadcasted_iota(jnp.int32, (128, 128), 1) % 16
    f = (a == b).astype(jnp.float32)
    return jnp.dot(st, f, preferred_element_type=jnp.float32)


def _bn_relu_packed(hp, st, g128, be128):
    s = _chanfold(st)
    mu = s[0:1, :] * (1.0 / N)
    var = s[1:2, :] * (1.0 / N) - mu * mu
    inv = lax.rsqrt(var + 1e-5)
    return jnp.maximum((hp - mu) * inv * g128 + be128, 0.0)


def _mm2_body(x_ref, w1_ref, wre_ref, tab_ref, xr_ref):
    xb = x_ref[...]
    tab_ref[0] = jnp.dot(xb, w1_ref[...], preferred_element_type=jnp.float32)

    @pl.when(pl.program_id(1) == 0)
    def _():
        xr_big = jnp.dot(xb, wre_ref[...], preferred_element_type=jnp.float32)
        xr_ref[...] = _pack(xr_big)


def _tc_mm2(x, w1, wre):
    k = x.shape[1]
    return pl.pallas_call(
        _mm2_body,
        grid=(_GRID, 2),
        in_specs=[
            pl.BlockSpec((_BLK, k), lambda i, j: (i, 0)),
            pl.BlockSpec((k, 128), lambda i, j: (0, j)),
            pl.BlockSpec((k, 128), lambda i, j: (0, 0)),
        ],
        out_specs=[
            pl.BlockSpec((1, _BLK, 128), lambda i, j: (j, i, 0)),
            pl.BlockSpec((_BLK8, 128), lambda i, j: (i, 0)),
        ],
        out_shape=[
            jax.ShapeDtypeStruct((2, NP, 128), jnp.float32),
            jax.ShapeDtypeStruct((MN8P, 128), jnp.float32),
        ],
    )(x, w1, wre)


def _stats_body(a_ref, xr_ref, b_ref, h_ref, s_ref):
    i = pl.program_id(0)
    h = a_ref[0] + a_ref[1] + xr_ref[...] + b_ref[...]
    row = lax.broadcasted_iota(jnp.int32, (_BLK8, 128), 0) + _BLK8 * i
    h = jnp.where(row < MN8, h, 0.0)
    h_ref[...] = h
    upd = jnp.concatenate([jnp.sum(h, 0)[None], jnp.sum(h * h, 0)[None]], 0)

    @pl.when(i == 0)
    def _():
        s_ref[...] = jnp.zeros_like(s_ref)

    s_ref[...] += upd


def _tc_stats(agg, xrp, b128):
    return pl.pallas_call(
        _stats_body,
        grid=(_GRID,),
        in_specs=[
            pl.BlockSpec((2, _BLK8, 128), lambda i: (0, i, 0)),
            pl.BlockSpec((_BLK8, 128), lambda i: (i, 0)),
            pl.BlockSpec((1, 128), lambda i: (0, 0)),
        ],
        out_specs=[
            pl.BlockSpec((_BLK8, 128), lambda i: (i, 0)),
            pl.BlockSpec((2, 128), lambda i: (0, 0)),
        ],
        out_shape=[
            jax.ShapeDtypeStruct((MN8P, 128), jnp.float32),
            jax.ShapeDtypeStruct((2, 128), jnp.float32),
        ],
    )(agg, xrp, b128)


def _norm_mm_body(h_ref, st_ref, g_ref, be_ref, wexp_ref, wrbd_ref,
                  tab_ref, hr_ref):
    hnp = _bn_relu_packed(h_ref[...], st_ref[...], g_ref[...], be_ref[...])
    hn_big = _unpack(hnp)
    tab_ref[0] = jnp.dot(hn_big, wexp_ref[...], preferred_element_type=jnp.float32)

    @pl.when(pl.program_id(1) == 0)
    def _():
        hr_ref[...] = jnp.dot(hnp, wrbd_ref[...], preferred_element_type=jnp.float32)


def _tc_norm_mm(h, st, g128, be128, wexp, wrbd):
    return pl.pallas_call(
        _norm_mm_body,
        grid=(_GRID, 2),
        in_specs=[
            pl.BlockSpec((_BLK8, 128), lambda i, j: (i, 0)),
            pl.BlockSpec((2, 128), lambda i, j: (0, 0)),
            pl.BlockSpec((1, 128), lambda i, j: (0, 0)),
            pl.BlockSpec((1, 128), lambda i, j: (0, 0)),
            pl.BlockSpec((128, 128), lambda i, j: (0, j)),
            pl.BlockSpec((128, 128), lambda i, j: (0, 0)),
        ],
        out_specs=[
            pl.BlockSpec((1, _BLK, 128), lambda i, j: (j, i, 0)),
            pl.BlockSpec((_BLK8, 128), lambda i, j: (i, 0)),
        ],
        out_shape=[
            jax.ShapeDtypeStruct((2, NP, 128), jnp.float32),
            jax.ShapeDtypeStruct((MN8P, 128), jnp.float32),
        ],
    )(h, st, g128, be128, wexp, wrbd)


def _final_body(h_ref, st_ref, g_ref, be_ref, wf_ref, bf_ref, o_ref):
    hnp = _bn_relu_packed(h_ref[...], st_ref[...], g_ref[...], be_ref[...])
    hn_big = _unpack(hnp)
    lg = jnp.dot(hn_big, wf_ref[...], preferred_element_type=jnp.float32) + bf_ref[...]
    m = jnp.max(lg, axis=1, keepdims=True)
    s = jnp.log(jnp.sum(jnp.exp(lg - m), axis=1, keepdims=True))
    o_ref[...] = lg - m - s


def _tc_final(h, st, g128, be128, wfexp, bf):
    return pl.pallas_call(
        _final_body,
        grid=(_GRID,),
        in_specs=[
            pl.BlockSpec((_BLK8, 128), lambda i: (i, 0)),
            pl.BlockSpec((2, 128), lambda i: (0, 0)),
            pl.BlockSpec((1, 128), lambda i: (0, 0)),
            pl.BlockSpec((1, 128), lambda i: (0, 0)),
            pl.BlockSpec((128, C), lambda i: (0, 0)),
            pl.BlockSpec((1, C), lambda i: (0, 0)),
        ],
        out_specs=pl.BlockSpec((_BLK, C), lambda i: (i, 0)),
        out_shape=jax.ShapeDtypeStruct((N, C), jnp.float32),
    )(h, st, g128, be128, wfexp, bf)


_MESH = plsc.VectorSubcoreMesh(core_axis_name="c", subcore_axis_name="s")
_SC_PARAMS = pltpu.CompilerParams(use_tc_tiling_on_sc=False)


def _keys_into(d_ref, e_ref, k2_ref):
    def body(i, _):
        dv = d_ref[pl.ds(16 * i, 16)]
        ev = e_ref[pl.ds(16 * i, 16)]
        cc = i // 8
        off = (i % 8) * 16
        k2_ref[cc, pl.ds(off, 16)] = dv * R + ev
        return 0
    lax.fori_loop(0, SUPER // 16, body, 0, unroll=4)


def _wnorm_body(dst_hbm, et_hbm, wn_hbm, counts_sh, d_buf, e_buf, k2, ones,
                cbuf, wbuf):
    s = lax.axis_index("s")
    c = lax.axis_index("c")

    def zb(i, _):
        cbuf[pl.ds(16 * i, 16)] = jnp.zeros((16,), jnp.float32)
        return 0
    lax.fori_loop(0, 125, zb, 0, unroll=4)

    def z2(j, _):
        pltpu.sync_copy(cbuf, counts_sh.at[pl.ds(s * 102000 + 2000 * j, 2000)])
        return 0
    lax.fori_loop(0, 51, z2, 0)

    def ob(i, _):
        ones[pl.ds(16 * i, 16)] = jnp.ones((16,), jnp.float32)
        return 0
    lax.fori_loop(0, CHUNK // 16, ob, 0)
    plsc.subcore_barrier()

    per_tile = E_PAD // NS

    def count_super(j, _):
        base = s * per_tile + j * SUPER
        pltpu.sync_copy(dst_hbm.at[pl.ds(base, SUPER)], d_buf)
        pltpu.sync_copy(et_hbm.at[pl.ds(base, SUPER)], e_buf)
        _keys_into(d_buf, e_buf, k2)

        def sc1(cc, _):
            pltpu.sync_copy(ones, counts_sh.at[k2.at[cc]], add=True)
            return 0
        lax.fori_loop(0, SUPER // CHUNK, sc1, 0)
        return 0
    lax.fori_loop(0, per_tile // SUPER, count_super, 0)
    plsc.subcore_barrier()

    def inv_chunk(j, _):
        off = s * 102000 + 2000 * j
        pltpu.sync_copy(counts_sh.at[pl.ds(off, 2000)], cbuf)

        def iv(i, _):
            v = cbuf[pl.ds(16 * i, 16)]
            cbuf[pl.ds(16 * i, 16)] = 1.0 / jnp.maximum(v, 1.0)
            return 0
        lax.fori_loop(0, 125, iv, 0, unroll=4)
        pltpu.sync_copy(cbuf, counts_sh.at[pl.ds(off, 2000)])
        return 0
    lax.fori_loop(0, 51, inv_chunk, 0)
    plsc.subcore_barrier()

    half = E_PAD // NC
    per_tile_h = half // NS

    def wn_super(j, _):
        base = c * half + s * per_tile_h + j * SUPER
        pltpu.sync_copy(dst_hbm.at[pl.ds(base, SUPER)], d_buf)
        pltpu.sync_copy(et_hbm.at[pl.ds(base, SUPER)], e_buf)
        _keys_into(d_buf, e_buf, k2)

        def g1(cc, _):
            pltpu.sync_copy(counts_sh.at[k2.at[cc]],
                            wbuf.at[pl.ds(CHUNK * cc, CHUNK)])
            return 0
        lax.fori_loop(0, SUPER // CHUNK, g1, 0)
        pltpu.sync_copy(wbuf, wn_hbm.at[pl.ds(base, SUPER)])
        return 0
    lax.fori_loop(0, per_tile_h // SUPER, wn_super, 0)


def _sc_wnorm(dstp, etp):
    fn = pl.kernel(
        _wnorm_body,
        out_type=jax.ShapeDtypeStruct((E_PAD,), jnp.float32),
        mesh=_MESH,
        scratch_types=[
            pltpu.VMEM_SHARED((CN,), jnp.float32),
            pltpu.VMEM((SUPER,), jnp.int32),
            pltpu.VMEM((SUPER,), jnp.int32),
            pltpu.VMEM((SUPER // CHUNK, CHUNK), jnp.int32),
            pltpu.VMEM((CHUNK,), jnp.float32),
            pltpu.VMEM((2000,), jnp.float32),
            pltpu.VMEM((SUPER,), jnp.float32),
        ],
        compiler_params=_SC_PARAMS,
    )
    return fn(dstp, etp)


def _msg_body(tab_hbm, src_hbm, et_hbm, wn_hbm, dst_hbm, agg_hbm,
              acc_sh,
              s_buf0, e_buf0, d_buf0, w_buf0, g_buf0, d20, rows0,
              s_buf1, e_buf1, d_buf1, w_buf1, g_buf1, d21, rows1,
              zrow, sem_st0, sem_st1, sem_g0, sem_g1, sem_sc0, sem_sc1):
    s = lax.axis_index("s")
    c = lax.axis_index("c")
    sb = (s_buf0, s_buf1)
    eb = (e_buf0, e_buf1)
    db = (d_buf0, d_buf1)
    wb = (w_buf0, w_buf1)
    gb = (g_buf0, g_buf1)
    d2 = (d20, d21)
    rows = (rows0, rows1)
    sem_st = (sem_st0, sem_st1)
    sem_g = (sem_g0, sem_g1)
    sem_sc = (sem_sc0, sem_sc1)
    nchunk = SUPER // CHUNK

    def zr(i, _):
        zrow[i, :] = jnp.zeros((16,), jnp.float32)
        return 0
    lax.fori_loop(0, CHUNK, zr, 0, unroll=4)

    def z2(j, _):
        pltpu.sync_copy(zrow, acc_sh.at[pl.ds(s * (ACC_ROWS // NS) + CHUNK * j, CHUNK), :])
        return 0
    lax.fori_loop(0, ACC_ROWS // NS // CHUNK, z2, 0)
    plsc.subcore_barrier()

    half = E_PAD // NC
    per_tile = half // NS
    nsup = per_tile // SUPER
    tile_base = c * half + s * per_tile

    def start_streams(p, base):
        pltpu.async_copy(src_hbm.at[pl.ds(base, SUPER)], sb[p], sem_st[p])
        pltpu.async_copy(et_hbm.at[pl.ds(base, SUPER)], eb[p], sem_st[p])
        pltpu.async_copy(dst_hbm.at[pl.ds(base, SUPER)], db[p], sem_st[p])
        pltpu.async_copy(wn_hbm.at[pl.ds(base, SUPER)], wb[p], sem_st[p])

    def wait_streams(p):
        pltpu.make_async_copy(src_hbm.at[pl.ds(0, SUPER)], sb[p], sem_st[p]).wait()
        pltpu.make_async_copy(et_hbm.at[pl.ds(0, SUPER)], eb[p], sem_st[p]).wait()
        pltpu.make_async_copy(dst_hbm.at[pl.ds(0, SUPER)], db[p], sem_st[p]).wait()
        pltpu.make_async_copy(wn_hbm.at[pl.ds(0, SUPER)], wb[p], sem_st[p]).wait()

    def wait_scatters(p):
        for cc in range(nchunk):
            pltpu.make_async_copy(rows[p].at[pl.ds(CHUNK * cc, CHUNK), :],
                                  acc_sh.at[d2[p].at[cc]], sem_sc[p]).wait()

    def process(p):
        def kb(i, _):
            sv = sb[p][pl.ds(16 * i, 16)]
            ev = eb[p][pl.ds(16 * i, 16)]
            gb[p][pl.ds(16 * i, 16)] = (sv * 8 + (ev & 7)
                                        + (ev >> 3) * (8 * NP))
            cc = i // 8
            off = (i % 8) * 16
            d2[p][cc, pl.ds(off, 16)] = db[p][pl.ds(16 * i, 16)]
            return 0
        lax.fori_loop(0, SUPER // 16, kb, 0, unroll=4)

        descs = []
        for cc in range(nchunk):
            descs.append(pltpu.async_copy(
                tab_hbm.at[gb[p].at[pl.ds(CHUNK * cc, CHUNK)]],
                rows[p].at[pl.ds(CHUNK * cc, CHUNK), :], sem_g[p]))
        for dsc in descs:
            dsc.wait()

        def sc_scale(i, _):
            wv = wb[p][pl.ds(16 * i, 16)]
            for k in range(16):
                rows[p][16 * i + k, :] = rows[p][16 * i + k, :] * wv[k]
            return 0
        lax.fori_loop(0, SUPER // 16, sc_scale, 0)

        for cc in range(nchunk):
            pltpu.async_copy(rows[p].at[pl.ds(CHUNK * cc, CHUNK), :],
                             acc_sh.at[d2[p].at[cc]], sem_sc[p], add=True)

    start_streams(0, tile_base)

    def pair(jj, _):
        j0 = 2 * jj
        wait_streams(0)
        start_streams(1, tile_base + (j0 + 1) * SUPER)

        @pl.when(jj > 0)
        def _():
            wait_scatters(0)
        process(0)

        wait_streams(1)

        @pl.when(j0 + 2 < nsup)
        def _():
            start_streams(0, tile_base + (j0 + 2) * SUPER)

        @pl.when(jj > 0)
        def _():
            wait_scatters(1)
        process(1)
        return 0
    lax.fori_loop(0, nsup // 2, pair, 0)
    wait_scatters(0)
    wait_scatters(1)
    plsc.subcore_barrier()

    def wo(j, _):
        k = j * NS + s

        @pl.when(k < ACC_ROWS // 1024)
        def _():
            r0 = k * 1024
            pltpu.sync_copy(acc_sh.at[pl.ds(r0, 1024), :],
                            agg_hbm.at[c, pl.ds(r0, 1024), :])
        return 0
    lax.fori_loop(0, (ACC_ROWS // 1024 + NS - 1) // NS, wo, 0)


def _sc_msg(table, srcp, etp, wnorm, dstp):
    per_parity = [
        pltpu.VMEM((SUPER,), jnp.int32),
        pltpu.VMEM((SUPER,), jnp.int32),
        pltpu.VMEM((SUPER,), jnp.int32),
        pltpu.VMEM((SUPER,), jnp.float32),
        pltpu.VMEM((SUPER,), jnp.int32),
        pltpu.VMEM((SUPER // CHUNK, CHUNK), jnp.int32),
        pltpu.VMEM((SUPER, H), jnp.float32),
    ]
    fn = pl.kernel(
        _msg_body,
        out_type=jax.ShapeDtypeStruct((NC, ACC_ROWS, H), jnp.float32),
        mesh=_MESH,
        scratch_types=(
            [pltpu.VMEM_SHARED((ACC_ROWS, H), jnp.float32)]
            + per_parity + per_parity
            + [pltpu.VMEM((CHUNK, H), jnp.float32)]
            + [pltpu.SemaphoreType.DMA] * 6
        ),
        compiler_params=_SC_PARAMS,
    )
    return fn(table, srcp, etp, wnorm, dstp)


def kernel(x, edge_index, edge_attr, W_rel0, W_root0, b0, gamma0, beta0,
           W_rel1, W_root1, b1, gamma1, beta1, Wf, bf):
    src = edge_index[0]
    dst = edge_index[1]
    et = edge_attr
    zpad = jnp.zeros((PADN,), jnp.int32)
    srcp = jnp.concatenate([src, zpad])
    etp = jnp.concatenate([et, zpad])
    dstp = jnp.concatenate([dst, jnp.full((PADN,), N, jnp.int32)])

    w1f0 = jnp.transpose(W_rel0, (1, 0, 2)).reshape(D, R * H)
    w1f1 = jnp.transpose(W_rel1, (1, 0, 2)).reshape(H, R * H)
    wre0 = jnp.tile(W_root0, (1, 8))
    wexp1 = jnp.tile(w1f1, (8, 1))
    wr1bd = jnp.kron(jnp.eye(8, dtype=jnp.float32), W_root1)
    wfexp = jnp.tile(Wf, (8, 1))
    b0_128 = jnp.tile(b0, 8).reshape(1, 128)
    g0_128 = jnp.tile(gamma0, 8).reshape(1, 128)
    be0_128 = jnp.tile(beta0, 8).reshape(1, 128)
    b1_128 = jnp.tile(b1, 8).reshape(1, 128)
    g1_128 = jnp.tile(gamma1, 8).reshape(1, 128)
    be1_128 = jnp.tile(beta1, 8).reshape(1, 128)

    tab0, xr0p = _tc_mm2(x, w1f0, wre0)
    wnorm = _sc_wnorm(dstp, etp)
    agg0 = _sc_msg(tab0.reshape(NP * R, H), srcp, etp, wnorm, dstp)
    h0p, st0 = _tc_stats(agg0.reshape(NC, MN8P, 128), xr0p, b0_128)
    tab1, hr1p = _tc_norm_mm(h0p, st0, g0_128, be0_128, wexp1, wr1bd)
    agg1 = _sc_msg(tab1.reshape(NP * R, H), srcp, etp, wnorm, dstp)
    h1p, st1 = _tc_stats(agg1.reshape(NC, MN8P, 128), hr1p, b1_128)
    return _tc_final(h1p, st1, g1_128, be1_128, wfexp, bf.reshape(1, C))


# speedup vs baseline: 14.2153x; 1.4694x over previous
import functools

import jax
import jax.numpy as jnp
from jax import lax
from jax.experimental import pallas as pl
from jax.experimental.pallas import tpu as pltpu
from jax.experimental.pallas import tpu_sc as plsc

N = 100000
E = 1600000
D = 128
H = 16
R = 16
C = 40

NC = 2
NS = 16
LANES = 16

SUPER = 512
CHUNK = 128
E_PAD = 1605632
PADN = E_PAD - E
CN = 1632000
ACC_ROWS = 100352

NP = 100352
_BLK = 1024
_BLK8 = _BLK // 8
MN8 = N // 8
MN8P = NP // 8
_GRID = NP // _BLK


def _bn_relu_packed(hp, st, g128, be128, fmat):
    s = jnp.dot(st, fmat, preferred_element_type=jnp.float32)
    mu = s[0:1, :] * (1.0 / N)
    var = s[1:2, :] * (1.0 / N) - mu * mu
    inv = lax.rsqrt(var + 1e-5)
    return jnp.maximum((hp - mu) * inv * g128 + be128, 0.0)


def _mm2_body(x_ref, w1_ref, wre_ref, pmat_ref, m8_ref, tab_ref, xr_ref):
    xb = x_ref[...]
    w1 = w1_ref[...]
    tab_ref[0] = jnp.dot(xb, w1[:, :128], preferred_element_type=jnp.float32)
    tab_ref[1] = jnp.dot(xb, w1[:, 128:], preferred_element_type=jnp.float32)
    xr_big = jnp.dot(xb, wre_ref[...], preferred_element_type=jnp.float32)
    xr_ref[...] = jnp.dot(pmat_ref[...], xr_big * m8_ref[...],
                          preferred_element_type=jnp.float32)


def _tc_mm2(x, w1, wre, pmat, m8):
    k = x.shape[1]
    return pl.pallas_call(
        _mm2_body,
        grid=(_GRID,),
        in_specs=[
            pl.BlockSpec((_BLK, k), lambda i: (i, 0)),
            pl.BlockSpec((k, 256), lambda i: (0, 0)),
            pl.BlockSpec((k, 128), lambda i: (0, 0)),
            pl.BlockSpec((_BLK8, _BLK), lambda i: (0, 0)),
            pl.BlockSpec((_BLK, 128), lambda i: (0, 0)),
        ],
        out_specs=[
            pl.BlockSpec((2, _BLK, 128), lambda i: (0, i, 0)),
            pl.BlockSpec((_BLK8, 128), lambda i: (i, 0)),
        ],
        out_shape=[
            jax.ShapeDtypeStruct((2, NP, 128), jnp.float32),
            jax.ShapeDtypeStruct((MN8P, 128), jnp.float32),
        ],
    )(x, w1, wre, pmat, m8)


def _stats_body(a_ref, xr_ref, b_ref, h_ref, s_ref):
    i = pl.program_id(0)
    h = a_ref[0] + a_ref[1] + xr_ref[...] + b_ref[...]
    row = lax.broadcasted_iota(jnp.int32, (_BLK8, 128), 0) + _BLK8 * i
    h = jnp.where(row < MN8, h, 0.0)
    h_ref[...] = h
    upd = jnp.concatenate([jnp.sum(h, 0)[None], jnp.sum(h * h, 0)[None]], 0)

    @pl.when(i == 0)
    def _():
        s_ref[...] = jnp.zeros_like(s_ref)

    s_ref[...] += upd


def _tc_stats(agg, xrp, b128):
    return pl.pallas_call(
        _stats_body,
        grid=(_GRID,),
        in_specs=[
            pl.BlockSpec((2, _BLK8, 128), lambda i: (0, i, 0)),
            pl.BlockSpec((_BLK8, 128), lambda i: (i, 0)),
            pl.BlockSpec((1, 128), lambda i: (0, 0)),
        ],
        out_specs=[
            pl.BlockSpec((_BLK8, 128), lambda i: (i, 0)),
            pl.BlockSpec((2, 128), lambda i: (0, 0)),
        ],
        out_shape=[
            jax.ShapeDtypeStruct((MN8P, 128), jnp.float32),
            jax.ShapeDtypeStruct((2, 128), jnp.float32),
        ],
    )(agg, xrp, b128)


def _norm_mm_body(h_ref, st_ref, g_ref, be_ref, wexp_ref, wrbd_ref,
                  tmat_ref, m8_ref, fmat_ref, tab_ref, hr_ref):
    hnp = _bn_relu_packed(h_ref[...], st_ref[...], g_ref[...], be_ref[...],
                          fmat_ref[...])
    hn_big = jnp.dot(tmat_ref[...], hnp,
                     preferred_element_type=jnp.float32) * m8_ref[...]
    wexp = wexp_ref[...]
    tab_ref[0] = jnp.dot(hn_big, wexp[:, :128], preferred_element_type=jnp.float32)
    tab_ref[1] = jnp.dot(hn_big, wexp[:, 128:], preferred_element_type=jnp.float32)
    hr_ref[...] = jnp.dot(hnp, wrbd_ref[...], preferred_element_type=jnp.float32)


def _tc_norm_mm(h, st, g128, be128, wexp, wrbd, tmat, m8, fmat):
    return pl.pallas_call(
        _norm_mm_body,
        grid=(_GRID,),
        in_specs=[
            pl.BlockSpec((_BLK8, 128), lambda i: (i, 0)),
            pl.BlockSpec((2, 128), lambda i: (0, 0)),
            pl.BlockSpec((1, 128), lambda i: (0, 0)),
            pl.BlockSpec((1, 128), lambda i: (0, 0)),
            pl.BlockSpec((128, 256), lambda i: (0, 0)),
            pl.BlockSpec((128, 128), lambda i: (0, 0)),
            pl.BlockSpec((_BLK, _BLK8), lambda i: (0, 0)),
            pl.BlockSpec((_BLK, 128), lambda i: (0, 0)),
            pl.BlockSpec((128, 128), lambda i: (0, 0)),
        ],
        out_specs=[
            pl.BlockSpec((2, _BLK, 128), lambda i: (0, i, 0)),
            pl.BlockSpec((_BLK8, 128), lambda i: (i, 0)),
        ],
        out_shape=[
            jax.ShapeDtypeStruct((2, NP, 128), jnp.float32),
            jax.ShapeDtypeStruct((MN8P, 128), jnp.float32),
        ],
    )(h, st, g128, be128, wexp, wrbd, tmat, m8, fmat)


def _final_body(h_ref, st_ref, g_ref, be_ref, wf_ref, bf_ref,
                tmat_ref, m8_ref, fmat_ref, o_ref):
    hnp = _bn_relu_packed(h_ref[...], st_ref[...], g_ref[...], be_ref[...],
                          fmat_ref[...])
    hn_big = jnp.dot(tmat_ref[...], hnp,
                     preferred_element_type=jnp.float32) * m8_ref[...]
    lg = jnp.dot(hn_big, wf_ref[...], preferred_element_type=jnp.float32) + bf_ref[...]
    m = jnp.max(lg, axis=1, keepdims=True)
    s = jnp.log(jnp.sum(jnp.exp(lg - m), axis=1, keepdims=True))
    o_ref[...] = lg - m - s


def _tc_final(h, st, g128, be128, wfexp, bf, tmat, m8, fmat):
    return pl.pallas_call(
        _final_body,
        grid=(_GRID,),
        in_specs=[
            pl.BlockSpec((_BLK8, 128), lambda i: (i, 0)),
            pl.BlockSpec((2, 128), lambda i: (0, 0)),
            pl.BlockSpec((1, 128), lambda i: (0, 0)),
            pl.BlockSpec((1, 128), lambda i: (0, 0)),
            pl.BlockSpec((128, C), lambda i: (0, 0)),
            pl.BlockSpec((1, C), lambda i: (0, 0)),
            pl.BlockSpec((_BLK, _BLK8), lambda i: (0, 0)),
            pl.BlockSpec((_BLK, 128), lambda i: (0, 0)),
            pl.BlockSpec((128, 128), lambda i: (0, 0)),
        ],
        out_specs=pl.BlockSpec((_BLK, C), lambda i: (i, 0)),
        out_shape=jax.ShapeDtypeStruct((N, C), jnp.float32),
    )(h, st, g128, be128, wfexp, bf, tmat, m8, fmat)


_MESH = plsc.VectorSubcoreMesh(core_axis_name="c", subcore_axis_name="s")
_SC_PARAMS = pltpu.CompilerParams(use_tc_tiling_on_sc=False)


WSUP = 1024
_N1 = E_PAD // NS
_N3 = E_PAD // NC // NS
_ICH = 2000


def _wnorm_body(dst_hbm, et_hbm, wn_hbm, counts_sh,
                d_buf0, e_buf0, k20, wbuf0,
                d_buf1, e_buf1, k21, wbuf1,
                ones, cbuf,
                sem_a0, sem_a1, sem_b0, sem_b1, sem_g):
    s = lax.axis_index("s")
    c = lax.axis_index("c")
    db = (d_buf0, d_buf1)
    eb = (e_buf0, e_buf1)
    k2 = (k20, k21)
    wb = (wbuf0, wbuf1)
    sem_a = (sem_a0, sem_a1)
    sem_b = (sem_b0, sem_b1)
    nch = WSUP // CHUNK

    def zb(i, _):
        cbuf[pl.ds(16 * i, 16)] = jnp.zeros((16,), jnp.float32)
        return 0
    lax.fori_loop(0, _ICH // 16, zb, 0, unroll=4)

    def ob(i, _):
        ones[pl.ds(16 * i, 16)] = jnp.ones((16,), jnp.float32)
        return 0
    lax.fori_loop(0, CHUNK // 16, ob, 0)

    def z2(j, _):
        pltpu.async_copy(cbuf, counts_sh.at[pl.ds(s * 102000 + _ICH * j, _ICH)],
                         sem_g)
        return 0
    lax.fori_loop(0, 102000 // _ICH, z2, 0)

    def z2w(j, _):
        pltpu.make_async_copy(cbuf, counts_sh.at[pl.ds(0, _ICH)], sem_g).wait()
        return 0
    lax.fori_loop(0, 102000 // _ICH, z2w, 0)
    plsc.subcore_barrier()

    def start_streams(p, base):
        pltpu.async_copy(dst_hbm.at[pl.ds(base, WSUP)], db[p], sem_a[p])
        pltpu.async_copy(et_hbm.at[pl.ds(base, WSUP)], eb[p], sem_a[p])

    def wait_streams(p):
        pltpu.make_async_copy(dst_hbm.at[pl.ds(0, WSUP)], db[p], sem_a[p]).wait()
        pltpu.make_async_copy(et_hbm.at[pl.ds(0, WSUP)], eb[p], sem_a[p]).wait()

    def keys(p):
        def kb(i, _):
            dv = db[p][pl.ds(16 * i, 16)]
            ev = eb[p][pl.ds(16 * i, 16)]
            cc = i // 8
            off = (i % 8) * 16
            k2[p][cc, pl.ds(off, 16)] = dv * R + ev
            return 0
        lax.fori_loop(0, WSUP // 16, kb, 0, unroll=4)

    def fire_count_scatters(p):
        for cc in range(nch):
            pltpu.async_copy(ones, counts_sh.at[k2[p].at[cc]], sem_b[p],
                             add=True)

    def drain_count_scatters(p):
        for cc in range(nch):
            pltpu.make_async_copy(ones, counts_sh.at[k2[p].at[cc]],
                                  sem_b[p]).wait()

    base1 = s * _N1
    start_streams(0, base1)

    def p1pair(jj, _):
        j0 = 2 * jj
        for p in (0, 1):
            j = j0 + p
            wait_streams(p)

            @pl.when(j + 1 < _N1 // WSUP)
            def _():
                start_streams(1 - p, base1 + (j + 1) * WSUP)

            @pl.when(jj > 0)
            def _():
                drain_count_scatters(p)
            keys(p)
            fire_count_scatters(p)
        return 0
    lax.fori_loop(0, _N1 // WSUP // 2, p1pair, 0)
    drain_count_scatters(0)
    drain_count_scatters(1)
    plsc.subcore_barrier()

    def inv_chunk(j, _):
        off = s * 102000 + _ICH * j
        pltpu.sync_copy(counts_sh.at[pl.ds(off, _ICH)], cbuf)

        def iv(i, _):
            v = cbuf[pl.ds(16 * i, 16)]
            cbuf[pl.ds(16 * i, 16)] = 1.0 / jnp.maximum(v, 1.0)
            return 0
        lax.fori_loop(0, _ICH // 16, iv, 0, unroll=4)
        pltpu.sync_copy(cbuf, counts_sh.at[pl.ds(off, _ICH)])
        return 0
    lax.fori_loop(0, 102000 // _ICH, inv_chunk, 0)
    plsc.subcore_barrier()

    def p3_step(p, j, first):
        base = c * (E_PAD // NC) + s * _N3 + j * WSUP
        wait_streams(p)

        @pl.when(j + 1 < _N3 // WSUP)
        def _():
            start_streams(1 - p, base + WSUP)

        @pl.when(jnp.logical_not(first))
        def _():
            pltpu.make_async_copy(wb[p], wn_hbm.at[pl.ds(0, WSUP)],
                                  sem_b[p]).wait()
        keys(p)
        for cc in range(nch):
            pltpu.async_copy(counts_sh.at[k2[p].at[cc]],
                             wb[p].at[pl.ds(CHUNK * cc, CHUNK)], sem_g)
        for cc in range(nch):
            pltpu.make_async_copy(counts_sh.at[k2[p].at[cc]],
                                  wb[p].at[pl.ds(CHUNK * cc, CHUNK)],
                                  sem_g).wait()
        pltpu.async_copy(wb[p], wn_hbm.at[pl.ds(base, WSUP)], sem_b[p])

    start_streams(0, c * (E_PAD // NC) + s * _N3)

    def p3pair(jj, _):
        p3_step(0, 2 * jj, jj == 0)
        p3_step(1, 2 * jj + 1, jj == 0)
        return 0
    lax.fori_loop(0, _N3 // WSUP // 2, p3pair, 0)
    p3_step(0, _N3 // WSUP - 1, False)
    pltpu.make_async_copy(wb[0], wn_hbm.at[pl.ds(0, WSUP)], sem_b[0]).wait()
    pltpu.make_async_copy(wb[1], wn_hbm.at[pl.ds(0, WSUP)], sem_b[1]).wait()


def _sc_wnorm(dstp, etp):
    fn = pl.kernel(
        _wnorm_body,
        out_type=jax.ShapeDtypeStruct((E_PAD,), jnp.float32),
        mesh=_MESH,
        scratch_types=(
            [pltpu.VMEM_SHARED((CN,), jnp.float32)]
            + [pltpu.VMEM((WSUP,), jnp.int32),
               pltpu.VMEM((WSUP,), jnp.int32),
               pltpu.VMEM((WSUP // CHUNK, CHUNK), jnp.int32),
               pltpu.VMEM((WSUP,), jnp.float32)] * 2
            + [pltpu.VMEM((CHUNK,), jnp.float32),
               pltpu.VMEM((_ICH,), jnp.float32)]
            + [pltpu.SemaphoreType.DMA] * 5
        ),
        compiler_params=_SC_PARAMS,
    )
    return fn(dstp, etp)


def _msg_body(tab_hbm, src_hbm, et_hbm, wn_hbm, dst_hbm, agg_hbm,
              acc_sh,
              s_buf0, e_buf0, d_buf0, w_buf0, g_buf0, d20, rows0,
              s_buf1, e_buf1, d_buf1, w_buf1, g_buf1, d21, rows1,
              zrow, sem_st0, sem_st1, sem_g0, sem_g1, sem_sc0, sem_sc1):
    s = lax.axis_index("s")
    c = lax.axis_index("c")
    sb = (s_buf0, s_buf1)
    eb = (e_buf0, e_buf1)
    db = (d_buf0, d_buf1)
    wb = (w_buf0, w_buf1)
    gb = (g_buf0, g_buf1)
    d2 = (d20, d21)
    rows = (rows0, rows1)
    sem_st = (sem_st0, sem_st1)
    sem_g = (sem_g0, sem_g1)
    sem_sc = (sem_sc0, sem_sc1)
    nchunk = SUPER // CHUNK

    def zr(i, _):
        zrow[i, :] = jnp.zeros((16,), jnp.float32)
        return 0
    lax.fori_loop(0, CHUNK, zr, 0, unroll=4)

    def z2(j, _):
        pltpu.async_copy(
            zrow, acc_sh.at[pl.ds(s * (ACC_ROWS // NS) + CHUNK * j, CHUNK), :],
            sem_g0)
        return 0
    lax.fori_loop(0, ACC_ROWS // NS // CHUNK, z2, 0)

    def z2w(j, _):
        pltpu.make_async_copy(zrow, acc_sh.at[pl.ds(0, CHUNK), :], sem_g0).wait()
        return 0
    lax.fori_loop(0, ACC_ROWS // NS // CHUNK, z2w, 0)
    plsc.subcore_barrier()

    half = E_PAD // NC
    per_tile = half // NS
    nsup = per_tile // SUPER
    tile_base = c * half + s * per_tile

    def start_streams(p, base):
        pltpu.async_copy(src_hbm.at[pl.ds(base, SUPER)], sb[p], sem_st[p])
        pltpu.async_copy(et_hbm.at[pl.ds(base, SUPER)], eb[p], sem_st[p])
        pltpu.async_copy(dst_hbm.at[pl.ds(base, SUPER)], db[p], sem_st[p])
        pltpu.async_copy(wn_hbm.at[pl.ds(base, SUPER)], wb[p], sem_st[p])

    def wait_streams(p):
        pltpu.make_async_copy(src_hbm.at[pl.ds(0, SUPER)], sb[p], sem_st[p]).wait()
        pltpu.make_async_copy(et_hbm.at[pl.ds(0, SUPER)], eb[p], sem_st[p]).wait()
        pltpu.make_async_copy(dst_hbm.at[pl.ds(0, SUPER)], db[p], sem_st[p]).wait()
        pltpu.make_async_copy(wn_hbm.at[pl.ds(0, SUPER)], wb[p], sem_st[p]).wait()

    def wait_scatters(p):
        for cc in range(nchunk):
            pltpu.make_async_copy(rows[p].at[pl.ds(CHUNK * cc, CHUNK), :],
                                  acc_sh.at[d2[p].at[cc]], sem_sc[p]).wait()

    def process(p):
        def kb(i, _):
            sv = sb[p][pl.ds(16 * i, 16)]
            ev = eb[p][pl.ds(16 * i, 16)]
            gb[p][pl.ds(16 * i, 16)] = (sv * 8 + (ev & 7)
                                        + (ev >> 3) * (8 * NP))
            cc = i // 8
            off = (i % 8) * 16
            d2[p][cc, pl.ds(off, 16)] = db[p][pl.ds(16 * i, 16)]
            return 0
        lax.fori_loop(0, SUPER // 16, kb, 0, unroll=4)

        descs = []
        for cc in range(nchunk):
            descs.append(pltpu.async_copy(
                tab_hbm.at[gb[p].at[pl.ds(CHUNK * cc, CHUNK)]],
                rows[p].at[pl.ds(CHUNK * cc, CHUNK), :], sem_g[p]))
        for dsc in descs:
            dsc.wait()

        def sc_scale(i, _):
            wv = wb[p][pl.ds(16 * i, 16)]
            for k in range(16):
                rows[p][16 * i + k, :] = rows[p][16 * i + k, :] * wv[k]
            return 0
        lax.fori_loop(0, SUPER // 16, sc_scale, 0)

        for cc in range(nchunk):
            pltpu.async_copy(rows[p].at[pl.ds(CHUNK * cc, CHUNK), :],
                             acc_sh.at[d2[p].at[cc]], sem_sc[p], add=True)

    start_streams(0, tile_base)

    def pair(jj, _):
        j0 = 2 * jj
        wait_streams(0)
        start_streams(1, tile_base + (j0 + 1) * SUPER)

        @pl.when(jj > 0)
        def _():
            wait_scatters(0)
        process(0)

        wait_streams(1)

        @pl.when(j0 + 2 < nsup)
        def _():
            start_streams(0, tile_base + (j0 + 2) * SUPER)

        @pl.when(jj > 0)
        def _():
            wait_scatters(1)
        process(1)
        return 0
    lax.fori_loop(0, nsup // 2, pair, 0)
    wait_scatters(0)
    wait_scatters(1)
    plsc.subcore_barrier()

    def wo(j, _):
        k = j * NS + s

        @pl.when(k < ACC_ROWS // 1024)
        def _():
            r0 = k * 1024
            pltpu.sync_copy(acc_sh.at[pl.ds(r0, 1024), :],
                            agg_hbm.at[c, pl.ds(r0, 1024), :])
        return 0
    lax.fori_loop(0, (ACC_ROWS // 1024 + NS - 1) // NS, wo, 0)


def _sc_msg(table, srcp, etp, wnorm, dstp):
    per_parity = [
        pltpu.VMEM((SUPER,), jnp.int32),
        pltpu.VMEM((SUPER,), jnp.int32),
        pltpu.VMEM((SUPER,), jnp.int32),
        pltpu.VMEM((SUPER,), jnp.float32),
        pltpu.VMEM((SUPER,), jnp.int32),
        pltpu.VMEM((SUPER // CHUNK, CHUNK), jnp.int32),
        pltpu.VMEM((SUPER, H), jnp.float32),
    ]
    fn = pl.kernel(
        _msg_body,
        out_type=jax.ShapeDtypeStruct((NC, ACC_ROWS, H), jnp.float32),
        mesh=_MESH,
        scratch_types=(
            [pltpu.VMEM_SHARED((ACC_ROWS, H), jnp.float32)]
            + per_parity + per_parity
            + [pltpu.VMEM((CHUNK, H), jnp.float32)]
            + [pltpu.SemaphoreType.DMA] * 6
        ),
        compiler_params=_SC_PARAMS,
    )
    return fn(table, srcp, etp, wnorm, dstp)


def kernel(x, edge_index, edge_attr, W_rel0, W_root0, b0, gamma0, beta0,
           W_rel1, W_root1, b1, gamma1, beta1, Wf, bf):
    ei_flat = edge_index.reshape(2 * E)
    src = ei_flat[:E]
    dst = ei_flat[E:]
    et = edge_attr
    zpad = jnp.zeros((PADN,), jnp.int32)
    srcp = jnp.concatenate([src, zpad])
    etp = jnp.concatenate([et, zpad])
    dstp = jnp.concatenate([dst, jnp.full((PADN,), N, jnp.int32)])

    w1f0 = jnp.transpose(W_rel0, (1, 0, 2)).reshape(D, R * H)
    w1f1 = jnp.transpose(W_rel1, (1, 0, 2)).reshape(H, R * H)
    wre0 = jnp.tile(W_root0, (1, 8))
    wexp1 = jnp.tile(w1f1, (8, 1))
    wr1bd = jnp.kron(jnp.eye(8, dtype=jnp.float32), W_root1)
    wfexp = jnp.tile(Wf, (8, 1))
    b0_128 = jnp.tile(b0, 8).reshape(1, 128)
    g0_128 = jnp.tile(gamma0, 8).reshape(1, 128)
    be0_128 = jnp.tile(beta0, 8).reshape(1, 128)
    b1_128 = jnp.tile(b1, 8).reshape(1, 128)
    g1_128 = jnp.tile(gamma1, 8).reshape(1, 128)
    be1_128 = jnp.tile(beta1, 8).reshape(1, 128)

    r_idx = jnp.arange(_BLK, dtype=jnp.int32)
    c_idx = jnp.arange(128, dtype=jnp.int32)
    m_idx = jnp.arange(_BLK8, dtype=jnp.int32)
    m8 = (r_idx[:, None] % 8 == c_idx[None, :] // 16).astype(jnp.float32)
    tmat = (r_idx[:, None] // 8 == m_idx[None, :]).astype(jnp.float32)
    pmat = tmat.T
    fmat = (c_idx[:, None] % 16 == c_idx[None, :] % 16).astype(jnp.float32)

    tab0, xr0p = _tc_mm2(x, w1f0, wre0, pmat, m8)
    wnorm = _sc_wnorm(dstp, etp)
    agg0 = _sc_msg(tab0.reshape(NP * R, H), srcp, etp, wnorm, dstp)
    h0p, st0 = _tc_stats(agg0.reshape(NC, MN8P, 128), xr0p, b0_128)
    tab1, hr1p = _tc_norm_mm(h0p, st0, g0_128, be0_128, wexp1, wr1bd,
                             tmat, m8, fmat)
    agg1 = _sc_msg(tab1.reshape(NP * R, H), srcp, etp, wnorm, dstp)
    h1p, st1 = _tc_stats(agg1.reshape(NC, MN8P, 128), hr1p, b1_128)
    return _tc_final(h1p, st1, g1_128, be1_128, wfexp, bf.reshape(1, C),
                     tmat, m8, fmat)


# speedup vs baseline: 16.0097x; 1.1262x over previous
import functools

import jax
import jax.numpy as jnp
from jax import lax
from jax.experimental import pallas as pl
from jax.experimental.pallas import tpu as pltpu
from jax.experimental.pallas import tpu_sc as plsc

N = 100000
E = 1600000
D = 128
H = 16
R = 16
C = 40

NC = 2
NS = 16
LANES = 16

SUPER = 512
CHUNK = 128
E_PAD = 1605632
PADN = E_PAD - E
CN = 1632000
ACC_ROWS = 100352

NP = 100352
_BLK = 2048
_BLK8 = _BLK // 8
MN8 = N // 8
MN8P = NP // 8
_GRID = NP // _BLK


def _bn_relu_packed(hp, st, g128, be128, fmat):
    s = jnp.dot(st, fmat, preferred_element_type=jnp.float32)
    mu = s[0:1, :] * (1.0 / N)
    var = s[1:2, :] * (1.0 / N) - mu * mu
    inv = lax.rsqrt(var + 1e-5)
    return jnp.maximum((hp - mu) * inv * g128 + be128, 0.0)


def _mm2_body(x_ref, w1_ref, wre_ref, pmat_ref, m8_ref, tab_ref, xr_ref):
    xb = x_ref[...]
    w1 = w1_ref[...]
    tab_ref[0] = jnp.dot(xb, w1[:, :128], preferred_element_type=jnp.float32)
    tab_ref[1] = jnp.dot(xb, w1[:, 128:], preferred_element_type=jnp.float32)
    xr_big = jnp.dot(xb, wre_ref[...], preferred_element_type=jnp.float32)
    xr_ref[...] = jnp.dot(pmat_ref[...], xr_big * m8_ref[...],
                          preferred_element_type=jnp.float32)


def _tc_mm2(x, w1, wre, pmat, m8):
    k = x.shape[1]
    return pl.pallas_call(
        _mm2_body,
        grid=(_GRID,),
        in_specs=[
            pl.BlockSpec((_BLK, k), lambda i: (i, 0)),
            pl.BlockSpec((k, 256), lambda i: (0, 0)),
            pl.BlockSpec((k, 128), lambda i: (0, 0)),
            pl.BlockSpec((_BLK8, _BLK), lambda i: (0, 0)),
            pl.BlockSpec((_BLK, 128), lambda i: (0, 0)),
        ],
        out_specs=[
            pl.BlockSpec((2, _BLK, 128), lambda i: (0, i, 0)),
            pl.BlockSpec((_BLK8, 128), lambda i: (i, 0)),
        ],
        out_shape=[
            jax.ShapeDtypeStruct((2, NP, 128), jnp.float32),
            jax.ShapeDtypeStruct((MN8P, 128), jnp.float32),
        ],
    )(x, w1, wre, pmat, m8)


def _stats_body(a_ref, xr_ref, b_ref, h_ref, s_ref):
    i = pl.program_id(0)
    h = a_ref[0] + a_ref[1] + xr_ref[...] + b_ref[...]
    row = lax.broadcasted_iota(jnp.int32, (_BLK8, 128), 0) + _BLK8 * i
    h = jnp.where(row < MN8, h, 0.0)
    h_ref[...] = h
    upd = jnp.concatenate([jnp.sum(h, 0)[None], jnp.sum(h * h, 0)[None]], 0)

    @pl.when(i == 0)
    def _():
        s_ref[...] = jnp.zeros_like(s_ref)

    s_ref[...] += upd


def _tc_stats(agg, xrp, b128):
    return pl.pallas_call(
        _stats_body,
        grid=(_GRID,),
        in_specs=[
            pl.BlockSpec((2, _BLK8, 128), lambda i: (0, i, 0)),
            pl.BlockSpec((_BLK8, 128), lambda i: (i, 0)),
            pl.BlockSpec((1, 128), lambda i: (0, 0)),
        ],
        out_specs=[
            pl.BlockSpec((_BLK8, 128), lambda i: (i, 0)),
            pl.BlockSpec((2, 128), lambda i: (0, 0)),
        ],
        out_shape=[
            jax.ShapeDtypeStruct((MN8P, 128), jnp.float32),
            jax.ShapeDtypeStruct((2, 128), jnp.float32),
        ],
    )(agg, xrp, b128)


def _norm_mm_body(h_ref, st_ref, g_ref, be_ref, wexp_ref, wrbd_ref,
                  tmat_ref, m8_ref, fmat_ref, tab_ref, hr_ref):
    hnp = _bn_relu_packed(h_ref[...], st_ref[...], g_ref[...], be_ref[...],
                          fmat_ref[...])
    hn_big = jnp.dot(tmat_ref[...], hnp,
                     preferred_element_type=jnp.float32) * m8_ref[...]
    wexp = wexp_ref[...]
    tab_ref[0] = jnp.dot(hn_big, wexp[:, :128], preferred_element_type=jnp.float32)
    tab_ref[1] = jnp.dot(hn_big, wexp[:, 128:], preferred_element_type=jnp.float32)
    hr_ref[...] = jnp.dot(hnp, wrbd_ref[...], preferred_element_type=jnp.float32)


def _tc_norm_mm(h, st, g128, be128, wexp, wrbd, tmat, m8, fmat):
    return pl.pallas_call(
        _norm_mm_body,
        grid=(_GRID,),
        in_specs=[
            pl.BlockSpec((_BLK8, 128), lambda i: (i, 0)),
            pl.BlockSpec((2, 128), lambda i: (0, 0)),
            pl.BlockSpec((1, 128), lambda i: (0, 0)),
            pl.BlockSpec((1, 128), lambda i: (0, 0)),
            pl.BlockSpec((128, 256), lambda i: (0, 0)),
            pl.BlockSpec((128, 128), lambda i: (0, 0)),
            pl.BlockSpec((_BLK, _BLK8), lambda i: (0, 0)),
            pl.BlockSpec((_BLK, 128), lambda i: (0, 0)),
            pl.BlockSpec((128, 128), lambda i: (0, 0)),
        ],
        out_specs=[
            pl.BlockSpec((2, _BLK, 128), lambda i: (0, i, 0)),
            pl.BlockSpec((_BLK8, 128), lambda i: (i, 0)),
        ],
        out_shape=[
            jax.ShapeDtypeStruct((2, NP, 128), jnp.float32),
            jax.ShapeDtypeStruct((MN8P, 128), jnp.float32),
        ],
    )(h, st, g128, be128, wexp, wrbd, tmat, m8, fmat)


def _final_body(h_ref, st_ref, g_ref, be_ref, wf_ref, bf_ref,
                tmat_ref, m8_ref, fmat_ref, o_ref):
    hnp = _bn_relu_packed(h_ref[...], st_ref[...], g_ref[...], be_ref[...],
                          fmat_ref[...])
    hn_big = jnp.dot(tmat_ref[...], hnp,
                     preferred_element_type=jnp.float32) * m8_ref[...]
    lg = jnp.dot(hn_big, wf_ref[...], preferred_element_type=jnp.float32) + bf_ref[...]
    m = jnp.max(lg, axis=1, keepdims=True)
    s = jnp.log(jnp.sum(jnp.exp(lg - m), axis=1, keepdims=True))
    o_ref[...] = lg - m - s


def _tc_final(h, st, g128, be128, wfexp, bf, tmat, m8, fmat):
    return pl.pallas_call(
        _final_body,
        grid=(_GRID,),
        in_specs=[
            pl.BlockSpec((_BLK8, 128), lambda i: (i, 0)),
            pl.BlockSpec((2, 128), lambda i: (0, 0)),
            pl.BlockSpec((1, 128), lambda i: (0, 0)),
            pl.BlockSpec((1, 128), lambda i: (0, 0)),
            pl.BlockSpec((128, C), lambda i: (0, 0)),
            pl.BlockSpec((1, C), lambda i: (0, 0)),
            pl.BlockSpec((_BLK, _BLK8), lambda i: (0, 0)),
            pl.BlockSpec((_BLK, 128), lambda i: (0, 0)),
            pl.BlockSpec((128, 128), lambda i: (0, 0)),
        ],
        out_specs=pl.BlockSpec((_BLK, C), lambda i: (i, 0)),
        out_shape=jax.ShapeDtypeStruct((N, C), jnp.float32),
    )(h, st, g128, be128, wfexp, bf, tmat, m8, fmat)


_MESH = plsc.VectorSubcoreMesh(core_axis_name="c", subcore_axis_name="s")
_SC_PARAMS = pltpu.CompilerParams(use_tc_tiling_on_sc=False)


WSUP = 1024
_N1 = E_PAD // NS
_N3 = E_PAD // NC // NS
_ICH = 2000


def _wnorm_body(dst_hbm, et_hbm, wn_hbm, counts_sh,
                d_buf0, e_buf0, k20, wbuf0,
                d_buf1, e_buf1, k21, wbuf1,
                ones, cbuf,
                sem_a0, sem_a1, sem_b0, sem_b1, sem_g):
    s = lax.axis_index("s")
    c = lax.axis_index("c")
    db = (d_buf0, d_buf1)
    eb = (e_buf0, e_buf1)
    k2 = (k20, k21)
    wb = (wbuf0, wbuf1)
    sem_a = (sem_a0, sem_a1)
    sem_b = (sem_b0, sem_b1)
    nch = WSUP // CHUNK

    def zb(i, _):
        cbuf[pl.ds(16 * i, 16)] = jnp.zeros((16,), jnp.float32)
        return 0
    lax.fori_loop(0, _ICH // 16, zb, 0, unroll=4)

    def ob(i, _):
        ones[pl.ds(16 * i, 16)] = jnp.ones((16,), jnp.float32)
        return 0
    lax.fori_loop(0, CHUNK // 16, ob, 0)

    def z2(j, _):
        pltpu.async_copy(cbuf, counts_sh.at[pl.ds(s * 102000 + _ICH * j, _ICH)],
                         sem_g)
        return 0
    lax.fori_loop(0, 102000 // _ICH, z2, 0)

    def z2w(j, _):
        pltpu.make_async_copy(cbuf, counts_sh.at[pl.ds(0, _ICH)], sem_g).wait()
        return 0
    lax.fori_loop(0, 102000 // _ICH, z2w, 0)
    plsc.subcore_barrier()

    def start_streams(p, base):
        pltpu.async_copy(dst_hbm.at[pl.ds(base, WSUP)], db[p], sem_a[p])
        pltpu.async_copy(et_hbm.at[pl.ds(base, WSUP)], eb[p], sem_a[p])

    def wait_streams(p):
        pltpu.make_async_copy(dst_hbm.at[pl.ds(0, WSUP)], db[p], sem_a[p]).wait()
        pltpu.make_async_copy(et_hbm.at[pl.ds(0, WSUP)], eb[p], sem_a[p]).wait()

    def keys(p):
        def kb(i, _):
            dv = db[p][pl.ds(16 * i, 16)]
            ev = eb[p][pl.ds(16 * i, 16)]
            cc = i // 8
            off = (i % 8) * 16
            k2[p][cc, pl.ds(off, 16)] = dv * R + ev
            return 0
        lax.fori_loop(0, WSUP // 16, kb, 0, unroll=4)

    def fire_count_scatters(p):
        for cc in range(nch):
            pltpu.async_copy(ones, counts_sh.at[k2[p].at[cc]], sem_b[p],
                             add=True)

    def drain_count_scatters(p):
        for cc in range(nch):
            pltpu.make_async_copy(ones, counts_sh.at[k2[p].at[cc]],
                                  sem_b[p]).wait()

    base1 = s * _N1
    start_streams(0, base1)

    def p1pair(jj, _):
        j0 = 2 * jj
        for p in (0, 1):
            j = j0 + p
            wait_streams(p)

            @pl.when(j + 1 < _N1 // WSUP)
            def _():
                start_streams(1 - p, base1 + (j + 1) * WSUP)

            @pl.when(jj > 0)
            def _():
                drain_count_scatters(p)
            keys(p)
            fire_count_scatters(p)
        return 0
    lax.fori_loop(0, _N1 // WSUP // 2, p1pair, 0)
    drain_count_scatters(0)
    drain_count_scatters(1)
    plsc.subcore_barrier()

    def inv_chunk(j, _):
        off = s * 102000 + _ICH * j
        pltpu.sync_copy(counts_sh.at[pl.ds(off, _ICH)], cbuf)

        def iv(i, _):
            v = cbuf[pl.ds(16 * i, 16)]
            cbuf[pl.ds(16 * i, 16)] = 1.0 / jnp.maximum(v, 1.0)
            return 0
        lax.fori_loop(0, _ICH // 16, iv, 0, unroll=4)
        pltpu.sync_copy(cbuf, counts_sh.at[pl.ds(off, _ICH)])
        return 0
    lax.fori_loop(0, 102000 // _ICH, inv_chunk, 0)
    plsc.subcore_barrier()

    def p3_step(p, j, first):
        base = c * (E_PAD // NC) + s * _N3 + j * WSUP
        wait_streams(p)

        @pl.when(j + 1 < _N3 // WSUP)
        def _():
            start_streams(1 - p, base + WSUP)

        @pl.when(jnp.logical_not(first))
        def _():
            pltpu.make_async_copy(wb[p], wn_hbm.at[pl.ds(0, WSUP)],
                                  sem_b[p]).wait()
        keys(p)
        for cc in range(nch):
            pltpu.async_copy(counts_sh.at[k2[p].at[cc]],
                             wb[p].at[pl.ds(CHUNK * cc, CHUNK)], sem_g)
        for cc in range(nch):
            pltpu.make_async_copy(counts_sh.at[k2[p].at[cc]],
                                  wb[p].at[pl.ds(CHUNK * cc, CHUNK)],
                                  sem_g).wait()
        pltpu.async_copy(wb[p], wn_hbm.at[pl.ds(base, WSUP)], sem_b[p])

    start_streams(0, c * (E_PAD // NC) + s * _N3)

    def p3pair(jj, _):
        p3_step(0, 2 * jj, jj == 0)
        p3_step(1, 2 * jj + 1, jj == 0)
        return 0
    lax.fori_loop(0, _N3 // WSUP // 2, p3pair, 0)
    p3_step(0, _N3 // WSUP - 1, False)
    pltpu.make_async_copy(wb[0], wn_hbm.at[pl.ds(0, WSUP)], sem_b[0]).wait()
    pltpu.make_async_copy(wb[1], wn_hbm.at[pl.ds(0, WSUP)], sem_b[1]).wait()


def _sc_wnorm(dstp, etp):
    fn = pl.kernel(
        _wnorm_body,
        out_type=jax.ShapeDtypeStruct((E_PAD,), jnp.float32),
        mesh=_MESH,
        scratch_types=(
            [pltpu.VMEM_SHARED((CN,), jnp.float32)]
            + [pltpu.VMEM((WSUP,), jnp.int32),
               pltpu.VMEM((WSUP,), jnp.int32),
               pltpu.VMEM((WSUP // CHUNK, CHUNK), jnp.int32),
               pltpu.VMEM((WSUP,), jnp.float32)] * 2
            + [pltpu.VMEM((CHUNK,), jnp.float32),
               pltpu.VMEM((_ICH,), jnp.float32)]
            + [pltpu.SemaphoreType.DMA] * 5
        ),
        compiler_params=_SC_PARAMS,
    )
    return fn(dstp, etp)


def _msg_body(tab_hbm, src_hbm, et_hbm, wn_hbm, dst_hbm, agg_hbm,
              acc_sh,
              s_buf0, e_buf0, d_buf0, w_buf0, g_buf0, d20, rows0,
              s_buf1, e_buf1, d_buf1, w_buf1, g_buf1, d21, rows1,
              zrow, sem_st0, sem_st1, sem_g0, sem_g1, sem_sc0, sem_sc1):
    s = lax.axis_index("s")
    c = lax.axis_index("c")
    sb = (s_buf0, s_buf1)
    eb = (e_buf0, e_buf1)
    db = (d_buf0, d_buf1)
    wb = (w_buf0, w_buf1)
    gb = (g_buf0, g_buf1)
    d2 = (d20, d21)
    rows = (rows0, rows1)
    sem_st = (sem_st0, sem_st1)
    sem_g = (sem_g0, sem_g1)
    sem_sc = (sem_sc0, sem_sc1)
    nchunk = SUPER // CHUNK

    def zr(i, _):
        zrow[i, :] = jnp.zeros((16,), jnp.float32)
        return 0
    lax.fori_loop(0, CHUNK, zr, 0, unroll=4)

    def z2(j, _):
        pltpu.async_copy(
            zrow, acc_sh.at[pl.ds(s * (ACC_ROWS // NS) + CHUNK * j, CHUNK), :],
            sem_g0)
        return 0
    lax.fori_loop(0, ACC_ROWS // NS // CHUNK, z2, 0)

    def z2w(j, _):
        pltpu.make_async_copy(zrow, acc_sh.at[pl.ds(0, CHUNK), :], sem_g0).wait()
        return 0
    lax.fori_loop(0, ACC_ROWS // NS // CHUNK, z2w, 0)
    plsc.subcore_barrier()

    half = E_PAD // NC
    per_tile = half // NS
    nsup = per_tile // SUPER
    tile_base = c * half + s * per_tile

    def start_streams(p, base):
        pltpu.async_copy(src_hbm.at[pl.ds(base, SUPER)], sb[p], sem_st[p])
        pltpu.async_copy(et_hbm.at[pl.ds(base, SUPER)], eb[p], sem_st[p])
        pltpu.async_copy(dst_hbm.at[pl.ds(base, SUPER)], db[p], sem_st[p])
        pltpu.async_copy(wn_hbm.at[pl.ds(base, SUPER)], wb[p], sem_st[p])

    def wait_streams(p):
        pltpu.make_async_copy(src_hbm.at[pl.ds(0, SUPER)], sb[p], sem_st[p]).wait()
        pltpu.make_async_copy(et_hbm.at[pl.ds(0, SUPER)], eb[p], sem_st[p]).wait()
        pltpu.make_async_copy(dst_hbm.at[pl.ds(0, SUPER)], db[p], sem_st[p]).wait()
        pltpu.make_async_copy(wn_hbm.at[pl.ds(0, SUPER)], wb[p], sem_st[p]).wait()

    def wait_scatters(p):
        for cc in range(nchunk):
            pltpu.make_async_copy(rows[p].at[pl.ds(CHUNK * cc, CHUNK), :],
                                  acc_sh.at[d2[p].at[cc]], sem_sc[p]).wait()

    def process(p):
        def kb(i, _):
            sv = sb[p][pl.ds(16 * i, 16)]
            ev = eb[p][pl.ds(16 * i, 16)]
            gb[p][pl.ds(16 * i, 16)] = (sv * 8 + (ev & 7)
                                        + (ev >> 3) * (8 * NP))
            cc = i // 8
            off = (i % 8) * 16
            d2[p][cc, pl.ds(off, 16)] = db[p][pl.ds(16 * i, 16)]
            return 0
        lax.fori_loop(0, SUPER // 16, kb, 0, unroll=4)

        descs = []
        for cc in range(nchunk):
            descs.append(pltpu.async_copy(
                tab_hbm.at[gb[p].at[pl.ds(CHUNK * cc, CHUNK)]],
                rows[p].at[pl.ds(CHUNK * cc, CHUNK), :], sem_g[p]))
        for dsc in descs:
            dsc.wait()

        def sc_scale(i, _):
            wv = wb[p][pl.ds(16 * i, 16)]
            for k in range(16):
                rows[p][16 * i + k, :] = rows[p][16 * i + k, :] * wv[k]
            return 0
        lax.fori_loop(0, SUPER // 16, sc_scale, 0)

        for cc in range(nchunk):
            pltpu.async_copy(rows[p].at[pl.ds(CHUNK * cc, CHUNK), :],
                             acc_sh.at[d2[p].at[cc]], sem_sc[p], add=True)

    start_streams(0, tile_base)

    def pair(jj, _):
        j0 = 2 * jj
        wait_streams(0)
        start_streams(1, tile_base + (j0 + 1) * SUPER)

        @pl.when(jj > 0)
        def _():
            wait_scatters(0)
        process(0)

        wait_streams(1)

        @pl.when(j0 + 2 < nsup)
        def _():
            start_streams(0, tile_base + (j0 + 2) * SUPER)

        @pl.when(jj > 0)
        def _():
            wait_scatters(1)
        process(1)
        return 0
    lax.fori_loop(0, nsup // 2, pair, 0)
    wait_scatters(0)
    wait_scatters(1)
    plsc.subcore_barrier()

    def wo(j, _):
        k = j * NS + s

        @pl.when(k < ACC_ROWS // 1024)
        def _():
            r0 = k * 1024
            pltpu.sync_copy(acc_sh.at[pl.ds(r0, 1024), :],
                            agg_hbm.at[c, pl.ds(r0, 1024), :])
        return 0
    lax.fori_loop(0, (ACC_ROWS // 1024 + NS - 1) // NS, wo, 0)


def _sc_msg(table, srcp, etp, wnorm, dstp):
    per_parity = [
        pltpu.VMEM((SUPER,), jnp.int32),
        pltpu.VMEM((SUPER,), jnp.int32),
        pltpu.VMEM((SUPER,), jnp.int32),
        pltpu.VMEM((SUPER,), jnp.float32),
        pltpu.VMEM((SUPER,), jnp.int32),
        pltpu.VMEM((SUPER // CHUNK, CHUNK), jnp.int32),
        pltpu.VMEM((SUPER, H), jnp.float32),
    ]
    fn = pl.kernel(
        _msg_body,
        out_type=jax.ShapeDtypeStruct((NC, ACC_ROWS, H), jnp.float32),
        mesh=_MESH,
        scratch_types=(
            [pltpu.VMEM_SHARED((ACC_ROWS, H), jnp.float32)]
            + per_parity + per_parity
            + [pltpu.VMEM((CHUNK, H), jnp.float32)]
            + [pltpu.SemaphoreType.DMA] * 6
        ),
        compiler_params=_SC_PARAMS,
    )
    return fn(table, srcp, etp, wnorm, dstp)


def kernel(x, edge_index, edge_attr, W_rel0, W_root0, b0, gamma0, beta0,
           W_rel1, W_root1, b1, gamma1, beta1, Wf, bf):
    ei_flat = edge_index.reshape(2 * E)
    src = ei_flat[:E]
    dst = ei_flat[E:]
    et = edge_attr
    zpad = jnp.zeros((PADN,), jnp.int32)
    srcp = jnp.concatenate([src, zpad])
    etp = jnp.concatenate([et, zpad])
    dstp = jnp.concatenate([dst, jnp.full((PADN,), N, jnp.int32)])

    w1f0 = jnp.transpose(W_rel0, (1, 0, 2)).reshape(D, R * H)
    w1f1 = jnp.transpose(W_rel1, (1, 0, 2)).reshape(H, R * H)
    wre0 = jnp.tile(W_root0, (1, 8))
    wexp1 = jnp.tile(w1f1, (8, 1))
    wr1bd = jnp.kron(jnp.eye(8, dtype=jnp.float32), W_root1)
    wfexp = jnp.tile(Wf, (8, 1))
    b0_128 = jnp.tile(b0, 8).reshape(1, 128)
    g0_128 = jnp.tile(gamma0, 8).reshape(1, 128)
    be0_128 = jnp.tile(beta0, 8).reshape(1, 128)
    b1_128 = jnp.tile(b1, 8).reshape(1, 128)
    g1_128 = jnp.tile(gamma1, 8).reshape(1, 128)
    be1_128 = jnp.tile(beta1, 8).reshape(1, 128)

    r_idx = jnp.arange(_BLK, dtype=jnp.int32)
    c_idx = jnp.arange(128, dtype=jnp.int32)
    m_idx = jnp.arange(_BLK8, dtype=jnp.int32)
    m8 = (r_idx[:, None] % 8 == c_idx[None, :] // 16).astype(jnp.float32)
    tmat = (r_idx[:, None] // 8 == m_idx[None, :]).astype(jnp.float32)
    pmat = tmat.T
    fmat = (c_idx[:, None] % 16 == c_idx[None, :] % 16).astype(jnp.float32)

    tab0, xr0p = _tc_mm2(x, w1f0, wre0, pmat, m8)
    wnorm = _sc_wnorm(dstp, etp)
    agg0 = _sc_msg(tab0.reshape(NP * R, H), srcp, etp, wnorm, dstp)
    h0p, st0 = _tc_stats(agg0.reshape(NC, MN8P, 128), xr0p, b0_128)
    tab1, hr1p = _tc_norm_mm(h0p, st0, g0_128, be0_128, wexp1, wr1bd,
                             tmat, m8, fmat)
    agg1 = _sc_msg(tab1.reshape(NP * R, H), srcp, etp, wnorm, dstp)
    h1p, st1 = _tc_stats(agg1.reshape(NC, MN8P, 128), hr1p, b1_128)
    return _tc_final(h1p, st1, g1_128, be1_128, wfexp, bf.reshape(1, C),
                     tmat, m8, fmat)
